# Initial kernel scaffold; baseline (speedup 1.0000x reference)
import jax
import jax.numpy as jnp
from jax.experimental import pallas as pl


def kernel(x, hmix_a, hmix_b, hmix_c, stem_W, stem_b, b1_W, b1_b, b2_W, b2_b, glob_W, glob_b, head1_W, head1_b, head2_W, head2_b, hp_thresh, hp_sharp, hp_scale):
    raise NotImplementedError("write your pallas kernel here")


# speedup vs baseline: 12.3168x; 12.3168x over previous
import functools

import jax
import jax.numpy as jnp
from jax import lax
from jax.experimental import pallas as pl
from jax.experimental.pallas import tpu as pltpu
from jax.experimental.pallas import tpu_sc as plsc

B, N, C_IN = 4, 4096, 4
K = 16
W0, W1, W2 = 64, 128, 256
NUM_CLASSES = 3

NC, NS = 2, 16
NW = NC * NS
PTS_PER_W = (B * N) // NW
GRP = 8
M_TILE = 256


def _prep_body(x_ref, a_ref, b_ref, c_ref, ws_ref, bs_ref,
               w1a_ref, w1d_ref, w1pn_ref, w1pp_ref, b1_ref,
               coords_ref, a1_ref, s1_ref):
    x = x_ref[0]
    a = a_ref[0, 0]
    b = b_ref[0, 0]
    c = c_ref[0, 0]
    z = a * x[:, 2:3] + b * x[:, 3:4] + c
    zeros = jnp.zeros((N, 5), jnp.float32)
    coords = jnp.concatenate([x[:, 0:2], z, zeros], axis=1)
    coords_ref[0] = coords
    f0 = jnp.maximum(jnp.dot(x, ws_ref[...],
                             preferred_element_type=jnp.float32, precision=lax.Precision.DEFAULT)
                     + bs_ref[...], 0.0)
    cp_p = jnp.dot(coords, w1pp_ref[...], preferred_element_type=jnp.float32, precision=lax.Precision.DEFAULT)
    cp_n = jnp.dot(coords, w1pn_ref[...], preferred_element_type=jnp.float32, precision=lax.Precision.DEFAULT)
    a1_ref[0] = (jnp.dot(f0, w1a_ref[...], preferred_element_type=jnp.float32, precision=lax.Precision.DEFAULT)
                 + cp_n + b1_ref[...])
    s1_ref[0] = (jnp.dot(f0, w1d_ref[...], preferred_element_type=jnp.float32, precision=lax.Precision.DEFAULT)
                 + cp_p)


def _prep_call(x, a, b, c, ws, bs, w1a, w1d, w1pn, w1pp, b1, interpret=False):
    full = lambda s: pl.BlockSpec(s, lambda i: (0,) * len(s))
    bspec = lambda s: pl.BlockSpec(s, lambda i: (i, 0, 0))
    return pl.pallas_call(
        _prep_body,
        grid=(B,),
        in_specs=[bspec((1, N, C_IN)), full((1, 1)), full((1, 1)), full((1, 1)),
                  full((C_IN, W0)), full((1, W0)),
                  full((W0, W1)), full((W0, W1)), full((8, W1)), full((8, W1)),
                  full((1, W1))],
        out_specs=[bspec((1, N, 8)), bspec((1, N, W1)), bspec((1, N, W1))],
        out_shape=[jax.ShapeDtypeStruct((B, N, 8), jnp.float32),
                   jax.ShapeDtypeStruct((B, N, W1), jnp.float32),
                   jax.ShapeDtypeStruct((B, N, W1), jnp.float32)],
        interpret=interpret,
    )(x, a, b, c, ws, bs, w1a, w1d, w1pn, w1pp, b1)


def _topk_body(ct_ref, cT_ref, idx_ref, *, prec):
    bi = pl.program_id(0)
    ct = ct_ref[0]
    cT = cT_ref[0]
    xx_t = jnp.sum(ct * ct, axis=1, keepdims=True)
    xx_f = jnp.sum(cT * cT, axis=0, keepdims=True)
    d = xx_t + xx_f - 2.0 * jnp.dot(ct, cT,
                                    preferred_element_type=jnp.float32, precision=prec)
    d = jnp.maximum(d, 0.0)
    col = lax.broadcasted_iota(jnp.int32, (M_TILE, N), 1)
    big_i = jnp.int32(1 << 30)
    inf = jnp.float32(3.0e38)
    base = bi * N
    for k in range(K):
        m = jnp.min(d, axis=1, keepdims=True)
        j = jnp.min(jnp.where(d == m, col, big_i), axis=1,
                    keepdims=True)
        idx_ref[0, :, k:k + 1] = j + base
        d = jnp.where(col == j, inf, d)


def _topk_call(coords_tiles, coords_T, interpret=False,
               prec=lax.Precision.DEFAULT):
    n_t = N // M_TILE
    return pl.pallas_call(
        functools.partial(_topk_body, prec=prec),
        grid=(B, n_t),
        in_specs=[pl.BlockSpec((1, M_TILE, 8), lambda bi, t: (bi * n_t + t, 0, 0)),
                  pl.BlockSpec((1, 8, N), lambda bi, t: (bi, 0, 0))],
        out_specs=pl.BlockSpec((1, M_TILE, K), lambda bi, t: (bi * n_t + t, 0, 0)),
        out_shape=jax.ShapeDtypeStruct((B * n_t, M_TILE, K), jnp.int32),
        interpret=interpret,
    )(coords_tiles, coords_T)


def _mid_body(f1_ref, coords_ref, w2a_ref, w2d_ref, w2pn_ref, w2pp_ref, b2_ref,
              a2_ref, s2_ref):
    f1 = f1_ref[0]
    coords = coords_ref[0]
    cp_p = jnp.dot(coords, w2pp_ref[...], preferred_element_type=jnp.float32, precision=lax.Precision.DEFAULT)
    cp_n = jnp.dot(coords, w2pn_ref[...], preferred_element_type=jnp.float32, precision=lax.Precision.DEFAULT)
    a2_ref[0] = (jnp.dot(f1, w2a_ref[...], preferred_element_type=jnp.float32, precision=lax.Precision.DEFAULT)
                 + cp_n + b2_ref[...])
    s2_ref[0] = (jnp.dot(f1, w2d_ref[...], preferred_element_type=jnp.float32, precision=lax.Precision.DEFAULT)
                 + cp_p)


def _mid_call(f1, coords, w2a, w2d, w2pn, w2pp, b2, interpret=False):
    full = lambda s: pl.BlockSpec(s, lambda i: (0,) * len(s))
    bspec = lambda s: pl.BlockSpec(s, lambda i: (i, 0, 0))
    return pl.pallas_call(
        _mid_body,
        grid=(B,),
        in_specs=[bspec((1, N, W1)), bspec((1, N, 8)),
                  full((W1, W2)), full((W1, W2)), full((8, W2)), full((8, W2)),
                  full((1, W2))],
        out_specs=[bspec((1, N, W2)), bspec((1, N, W2))],
        out_shape=[jax.ShapeDtypeStruct((B, N, W2), jnp.float32),
                   jax.ShapeDtypeStruct((B, N, W2), jnp.float32)],
        interpret=interpret,
    )(f1, coords, w2a, w2d, w2pn, w2pp, b2)


@functools.cache
def _make_gather_max(D):
    mesh = plsc.VectorSubcoreMesh(core_axis_name="c", subcore_axis_name="s")
    n_grp = PTS_PER_W // GRP

    @functools.partial(
        pl.kernel, mesh=mesh,
        out_type=jax.ShapeDtypeStruct((B * N, D), jnp.float32),
        scratch_types=[
            pltpu.VMEM((GRP * K,), jnp.int32),
            pltpu.VMEM((GRP * K, D), jnp.float32),
            pltpu.VMEM((GRP, D), jnp.float32),
            pltpu.VMEM((GRP, D), jnp.float32),
            pltpu.SemaphoreType.DMA,
        ],
    )
    def gm(s_hbm, a_hbm, idx_hbm, out_hbm, idx_v, rows_v, a_v, out_v, sem):
        wid = lax.axis_index("s") * NC + lax.axis_index("c")
        base = wid * PTS_PER_W

        def body(g, _):
            row0 = base + g * GRP
            pltpu.sync_copy(idx_hbm.at[pl.ds(row0 * K, GRP * K)], idx_v)
            pltpu.async_copy(s_hbm.at[idx_v], rows_v, sem).wait()
            pltpu.sync_copy(a_hbm.at[pl.ds(row0, GRP)], a_v)
            for p in range(GRP):
                for dc in range(D // 16):
                    sl = pl.ds(dc * 16, 16)
                    acc = rows_v[p * K, sl]
                    for j in range(1, K):
                        acc = jnp.maximum(acc, rows_v[p * K + j, sl])
                    out_v[p, sl] = jnp.maximum(acc + a_v[p, sl], 0.0)
            pltpu.sync_copy(out_v, out_hbm.at[pl.ds(row0, GRP)])
            return _

        lax.fori_loop(0, n_grp, body, None)

    return gm


def _head_body(f2_ref, x_ref, gw_ref, gb_ref, hwf_ref, hwg_ref, hb_ref,
               h2w_ref, h2b_ref, th_ref, sh_ref, sc_ref, out_ref):
    f2 = f2_ref[0]
    g = jnp.max(f2, axis=0, keepdims=True)
    g = jnp.maximum(jnp.dot(g, gw_ref[...],
                            preferred_element_type=jnp.float32, precision=lax.Precision.DEFAULT)
                    + gb_ref[...], 0.0)
    gh = jnp.dot(g, hwg_ref[...], preferred_element_type=jnp.float32, precision=lax.Precision.DEFAULT) \
        + hb_ref[...]
    h = jnp.maximum(jnp.dot(f2, hwf_ref[...],
                            preferred_element_type=jnp.float32, precision=lax.Precision.DEFAULT) + gh, 0.0)
    logits = jnp.dot(h, h2w_ref[...],
                     preferred_element_type=jnp.float32, precision=lax.Precision.DEFAULT) + h2b_ref[...]
    hag = x_ref[0][:, 3:4]
    t = sh_ref[0, 0] * (th_ref[0, 0] - hag)
    bias = sc_ref[0, 0] / (1.0 + jnp.exp(-t))
    cls = lax.broadcasted_iota(jnp.int32, (N, NUM_CLASSES), 1)
    out_ref[0] = jnp.where(cls == 0, logits + bias, logits)


def _head_call(f2, x, gw, gb, hwf, hwg, hb, h2w, h2b, th, sh, sc,
               interpret=False):
    full = lambda s: pl.BlockSpec(s, lambda i: (0,) * len(s))
    bspec = lambda s: pl.BlockSpec(s, lambda i: (i, 0, 0))
    return pl.pallas_call(
        _head_body,
        grid=(B,),
        in_specs=[bspec((1, N, W2)), bspec((1, N, C_IN)),
                  full((W2, W2)), full((1, W2)),
                  full((W2, W2)), full((W2, W2)), full((1, W2)),
                  full((W2, NUM_CLASSES)), full((1, NUM_CLASSES)),
                  full((1, 1)), full((1, 1)), full((1, 1))],
        out_specs=bspec((1, N, NUM_CLASSES)),
        out_shape=jax.ShapeDtypeStruct((B, N, NUM_CLASSES), jnp.float32),
        interpret=interpret,
    )(f2, x, gw, gb, hwf, hwg, hb, h2w, h2b, th, sh, sc)


def kernel(x, hmix_a, hmix_b, hmix_c, stem_W, stem_b, b1_W, b1_b,
           b2_W, b2_b, glob_W, glob_b, head1_W, head1_b, head2_W, head2_b,
           hp_thresh, hp_sharp, hp_scale):
    f32 = jnp.float32
    s11 = lambda v: jnp.asarray(v, f32).reshape(1, 1)
    pad8 = lambda w: jnp.concatenate(
        [w, jnp.zeros((8 - w.shape[0], w.shape[1]), f32)], axis=0)

    w1f, w1d, w1p = b1_W[:W0], b1_W[W0:2 * W0], b1_W[2 * W0:]
    w2f, w2d, w2p = b2_W[:W1], b2_W[W1:2 * W1], b2_W[2 * W1:]

    coords, a1, s1 = _prep_call(
        x, s11(hmix_a), s11(hmix_b), s11(hmix_c),
        stem_W, stem_b.reshape(1, W0),
        w1f - w1d, w1d, pad8(-w1p), pad8(w1p), b1_b.reshape(1, W1))

    coords_T = jnp.swapaxes(coords, 1, 2)
    ct_tiles = coords.reshape(B * (N // M_TILE), M_TILE, 8)
    idx = _topk_call(ct_tiles, coords_T)
    idx_flat = idx.reshape(B * N * K)

    f1 = _make_gather_max(W1)(s1.reshape(B * N, W1), a1.reshape(B * N, W1),
                              idx_flat)

    a2, s2 = _mid_call(f1.reshape(B, N, W1), coords,
                       w2f - w2d, w2d, pad8(-w2p), pad8(w2p),
                       b2_b.reshape(1, W2))

    f2 = _make_gather_max(W2)(s2.reshape(B * N, W2), a2.reshape(B * N, W2),
                              idx_flat)

    return _head_call(f2.reshape(B, N, W2), x,
                      glob_W, glob_b.reshape(1, W2),
                      head1_W[:W2], head1_W[W2:], head1_b.reshape(1, W2),
                      head2_W, head2_b.reshape(1, NUM_CLASSES),
                      s11(hp_thresh), s11(hp_sharp), s11(hp_scale))


# speedup vs baseline: 14.4006x; 1.1692x over previous
import functools

import jax
import jax.numpy as jnp
from jax import lax
from jax.experimental import pallas as pl
from jax.experimental.pallas import tpu as pltpu
from jax.experimental.pallas import tpu_sc as plsc

B, N, C_IN = 4, 4096, 4
K = 16
W0, W1, W2 = 64, 128, 256
NUM_CLASSES = 3

NC, NS = 2, 16
NW = NC * NS
PTS_PER_W = (B * N) // NW
GRP = 8
M_TILE = 256


def _prep_body(x_ref, a_ref, b_ref, c_ref, ws_ref, bs_ref,
               w1a_ref, w1d_ref, w1pn_ref, w1pp_ref, b1_ref,
               coords_ref, a1_ref, s1_ref):
    x = x_ref[0]
    a = a_ref[0, 0]
    b = b_ref[0, 0]
    c = c_ref[0, 0]
    z = a * x[:, 2:3] + b * x[:, 3:4] + c
    zeros = jnp.zeros((N, 5), jnp.float32)
    coords = jnp.concatenate([x[:, 0:2], z, zeros], axis=1)
    coords_ref[0] = coords
    f0 = jnp.maximum(jnp.dot(x, ws_ref[...],
                             preferred_element_type=jnp.float32, precision=lax.Precision.DEFAULT)
                     + bs_ref[...], 0.0)
    cp_p = jnp.dot(coords, w1pp_ref[...], preferred_element_type=jnp.float32, precision=lax.Precision.DEFAULT)
    cp_n = jnp.dot(coords, w1pn_ref[...], preferred_element_type=jnp.float32, precision=lax.Precision.DEFAULT)
    a1_ref[0] = (jnp.dot(f0, w1a_ref[...], preferred_element_type=jnp.float32, precision=lax.Precision.DEFAULT)
                 + cp_n + b1_ref[...])
    s1_ref[0] = (jnp.dot(f0, w1d_ref[...], preferred_element_type=jnp.float32, precision=lax.Precision.DEFAULT)
                 + cp_p)


def _prep_call(x, a, b, c, ws, bs, w1a, w1d, w1pn, w1pp, b1, interpret=False):
    full = lambda s: pl.BlockSpec(s, lambda i: (0,) * len(s))
    bspec = lambda s: pl.BlockSpec(s, lambda i: (i, 0, 0))
    return pl.pallas_call(
        _prep_body,
        grid=(B,),
        in_specs=[bspec((1, N, C_IN)), full((1, 1)), full((1, 1)), full((1, 1)),
                  full((C_IN, W0)), full((1, W0)),
                  full((W0, W1)), full((W0, W1)), full((8, W1)), full((8, W1)),
                  full((1, W1))],
        out_specs=[bspec((1, N, 8)), bspec((1, N, W1)), bspec((1, N, W1))],
        out_shape=[jax.ShapeDtypeStruct((B, N, 8), jnp.float32),
                   jax.ShapeDtypeStruct((B, N, W1), jnp.float32),
                   jax.ShapeDtypeStruct((B, N, W1), jnp.float32)],
        interpret=interpret,
    )(x, a, b, c, ws, bs, w1a, w1d, w1pn, w1pp, b1)


def _topk_body(ct_ref, cT_ref, idx_ref, *, prec):
    bi = pl.program_id(0)
    ct = ct_ref[0]
    cT = cT_ref[0]
    xx_t = jnp.sum(ct * ct, axis=1, keepdims=True)
    xx_f = jnp.sum(cT * cT, axis=0, keepdims=True)
    d = xx_t + xx_f - 2.0 * jnp.dot(ct, cT,
                                    preferred_element_type=jnp.float32, precision=prec)
    d = jnp.maximum(d, 0.0)
    col = lax.broadcasted_iota(jnp.int32, (M_TILE, N), 1)
    big_i = jnp.int32(1 << 30)
    inf = jnp.float32(3.0e38)
    base = bi * N
    for k in range(K):
        m = jnp.min(d, axis=1, keepdims=True)
        j = jnp.min(jnp.where(d == m, col, big_i), axis=1,
                    keepdims=True)
        idx_ref[0, :, k:k + 1] = j + base
        d = jnp.where(col == j, inf, d)


def _topk_call(coords_tiles, coords_T, interpret=False,
               prec=lax.Precision.DEFAULT):
    n_t = N // M_TILE
    return pl.pallas_call(
        functools.partial(_topk_body, prec=prec),
        grid=(B, n_t),
        in_specs=[pl.BlockSpec((1, M_TILE, 8), lambda bi, t: (bi * n_t + t, 0, 0)),
                  pl.BlockSpec((1, 8, N), lambda bi, t: (bi, 0, 0))],
        out_specs=pl.BlockSpec((1, M_TILE, K), lambda bi, t: (bi * n_t + t, 0, 0)),
        out_shape=jax.ShapeDtypeStruct((B * n_t, M_TILE, K), jnp.int32),
        interpret=interpret,
    )(coords_tiles, coords_T)


def _mid_body(a1_ref, m1_ref, coords_ref, w2a_ref, w2d_ref, w2pn_ref,
              w2pp_ref, b2_ref, a2_ref, s2_ref):
    f1 = jnp.maximum(a1_ref[0] + m1_ref[0], 0.0)
    coords = coords_ref[0]
    cp_p = jnp.dot(coords, w2pp_ref[...], preferred_element_type=jnp.float32, precision=lax.Precision.DEFAULT)
    cp_n = jnp.dot(coords, w2pn_ref[...], preferred_element_type=jnp.float32, precision=lax.Precision.DEFAULT)
    a2_ref[0] = (jnp.dot(f1, w2a_ref[...], preferred_element_type=jnp.float32, precision=lax.Precision.DEFAULT)
                 + cp_n + b2_ref[...])
    s2_ref[0] = (jnp.dot(f1, w2d_ref[...], preferred_element_type=jnp.float32, precision=lax.Precision.DEFAULT)
                 + cp_p)


def _mid_call(a1, m1, coords, w2a, w2d, w2pn, w2pp, b2, interpret=False):
    full = lambda s: pl.BlockSpec(s, lambda i: (0,) * len(s))
    bspec = lambda s: pl.BlockSpec(s, lambda i: (i, 0, 0))
    return pl.pallas_call(
        _mid_body,
        grid=(B,),
        in_specs=[bspec((1, N, W1)), bspec((1, N, W1)), bspec((1, N, 8)),
                  full((W1, W2)), full((W1, W2)), full((8, W2)), full((8, W2)),
                  full((1, W2))],
        out_specs=[bspec((1, N, W2)), bspec((1, N, W2))],
        out_shape=[jax.ShapeDtypeStruct((B, N, W2), jnp.float32),
                   jax.ShapeDtypeStruct((B, N, W2), jnp.float32)],
        interpret=interpret,
    )(a1, m1, coords, w2a, w2d, w2pn, w2pp, b2)


@functools.cache
def _make_gather_max(D):
    mesh = plsc.VectorSubcoreMesh(core_axis_name="c", subcore_axis_name="s")
    n_grp = PTS_PER_W // GRP

    @functools.partial(
        pl.kernel, mesh=mesh,
        out_type=jax.ShapeDtypeStruct((B * N, D), jnp.float32),
        scratch_types=[
            pltpu.VMEM((PTS_PER_W * K,), jnp.int32),
            pltpu.VMEM((GRP * K, D), jnp.float32),
            pltpu.VMEM((GRP * K, D), jnp.float32),
            pltpu.VMEM((GRP, D), jnp.float32),
            pltpu.SemaphoreType.DMA,
            pltpu.SemaphoreType.DMA,
        ],
    )
    def gm(s_hbm, idx_hbm, out_hbm, idx_v, r0, r1, out_v, s0, s1):
        wid = lax.axis_index("s") * NC + lax.axis_index("c")
        base = wid * PTS_PER_W
        pltpu.sync_copy(idx_hbm.at[pl.ds(base * K, PTS_PER_W * K)], idx_v)
        bufs, sems = (r0, r1), (s0, s1)

        def gather(g, buf, sem):
            ii = idx_v.at[pl.ds(g * (GRP * K), GRP * K)]
            return pltpu.make_async_copy(s_hbm.at[ii], buf, sem)

        gather(0, r0, s0).start()
        gather(1, r1, s1).start()

        def body(i, _):
            for b in range(2):
                g = 2 * i + b
                buf, sem = bufs[b], sems[b]
                gather(g, buf, sem).wait()
                for p in range(GRP):
                    for dc in range(D // 16):
                        sl = pl.ds(dc * 16, 16)
                        acc = buf[p * K, sl]
                        for j in range(1, K):
                            acc = jnp.maximum(acc, buf[p * K + j, sl])
                        out_v[p, sl] = acc
                pltpu.sync_copy(out_v, out_hbm.at[pl.ds(base + g * GRP, GRP)])
                gather(jnp.minimum(g + 2, n_grp - 1), buf, sem).start()
            return _

        lax.fori_loop(0, n_grp // 2, body, None)
        gather(n_grp - 1, r0, s0).wait()
        gather(n_grp - 1, r1, s1).wait()

    return gm


def _head_body(a2_ref, m2_ref, x_ref, gw_ref, gb_ref, hwf_ref, hwg_ref,
               hb_ref, h2w_ref, h2b_ref, th_ref, sh_ref, sc_ref, out_ref):
    f2 = jnp.maximum(a2_ref[0] + m2_ref[0], 0.0)
    g = jnp.max(f2, axis=0, keepdims=True)
    g = jnp.maximum(jnp.dot(g, gw_ref[...],
                            preferred_element_type=jnp.float32, precision=lax.Precision.DEFAULT)
                    + gb_ref[...], 0.0)
    gh = jnp.dot(g, hwg_ref[...], preferred_element_type=jnp.float32, precision=lax.Precision.DEFAULT) \
        + hb_ref[...]
    h = jnp.maximum(jnp.dot(f2, hwf_ref[...],
                            preferred_element_type=jnp.float32, precision=lax.Precision.DEFAULT) + gh, 0.0)
    logits = jnp.dot(h, h2w_ref[...],
                     preferred_element_type=jnp.float32, precision=lax.Precision.DEFAULT) + h2b_ref[...]
    hag = x_ref[0][:, 3:4]
    t = sh_ref[0, 0] * (th_ref[0, 0] - hag)
    bias = sc_ref[0, 0] / (1.0 + jnp.exp(-t))
    cls = lax.broadcasted_iota(jnp.int32, (N, NUM_CLASSES), 1)
    out_ref[0] = jnp.where(cls == 0, logits + bias, logits)


def _head_call(a2, m2, x, gw, gb, hwf, hwg, hb, h2w, h2b, th, sh, sc,
               interpret=False):
    full = lambda s: pl.BlockSpec(s, lambda i: (0,) * len(s))
    bspec = lambda s: pl.BlockSpec(s, lambda i: (i, 0, 0))
    return pl.pallas_call(
        _head_body,
        grid=(B,),
        in_specs=[bspec((1, N, W2)), bspec((1, N, W2)), bspec((1, N, C_IN)),
                  full((W2, W2)), full((1, W2)),
                  full((W2, W2)), full((W2, W2)), full((1, W2)),
                  full((W2, NUM_CLASSES)), full((1, NUM_CLASSES)),
                  full((1, 1)), full((1, 1)), full((1, 1))],
        out_specs=bspec((1, N, NUM_CLASSES)),
        out_shape=jax.ShapeDtypeStruct((B, N, NUM_CLASSES), jnp.float32),
        interpret=interpret,
    )(a2, m2, x, gw, gb, hwf, hwg, hb, h2w, h2b, th, sh, sc)


def kernel(x, hmix_a, hmix_b, hmix_c, stem_W, stem_b, b1_W, b1_b,
           b2_W, b2_b, glob_W, glob_b, head1_W, head1_b, head2_W, head2_b,
           hp_thresh, hp_sharp, hp_scale):
    f32 = jnp.float32
    s11 = lambda v: jnp.asarray(v, f32).reshape(1, 1)
    pad8 = lambda w: jnp.concatenate(
        [w, jnp.zeros((8 - w.shape[0], w.shape[1]), f32)], axis=0)

    w1f, w1d, w1p = b1_W[:W0], b1_W[W0:2 * W0], b1_W[2 * W0:]
    w2f, w2d, w2p = b2_W[:W1], b2_W[W1:2 * W1], b2_W[2 * W1:]

    coords, a1, s1 = _prep_call(
        x, s11(hmix_a), s11(hmix_b), s11(hmix_c),
        stem_W, stem_b.reshape(1, W0),
        w1f - w1d, w1d, pad8(-w1p), pad8(w1p), b1_b.reshape(1, W1))

    coords_T = jnp.swapaxes(coords, 1, 2)
    ct_tiles = coords.reshape(B * (N // M_TILE), M_TILE, 8)
    idx = _topk_call(ct_tiles, coords_T)
    idx_flat = idx.reshape(B * N * K)

    m1 = _make_gather_max(W1)(s1.reshape(B * N, W1), idx_flat)

    a2, s2 = _mid_call(a1, m1.reshape(B, N, W1), coords,
                       w2f - w2d, w2d, pad8(-w2p), pad8(w2p),
                       b2_b.reshape(1, W2))

    m2 = _make_gather_max(W2)(s2.reshape(B * N, W2), idx_flat)

    return _head_call(a2, m2.reshape(B, N, W2), x,
                      glob_W, glob_b.reshape(1, W2),
                      head1_W[:W2], head1_W[W2:], head1_b.reshape(1, W2),
                      head2_W, head2_b.reshape(1, NUM_CLASSES),
                      s11(hp_thresh), s11(hp_sharp), s11(hp_scale))


# speedup vs baseline: 14.4269x; 1.0018x over previous
import functools

import jax
import jax.numpy as jnp
from jax import lax
from jax.experimental import pallas as pl
from jax.experimental.pallas import tpu as pltpu
from jax.experimental.pallas import tpu_sc as plsc

B, N, C_IN = 4, 4096, 4
K = 16
W0, W1, W2 = 64, 128, 256
NUM_CLASSES = 3

NC, NS = 2, 16
NW = NC * NS
PTS_PER_W = (B * N) // NW
GRP = 8
M_TILE = 256


def _prep_body(x_ref, a_ref, b_ref, c_ref, ws_ref, bs_ref,
               w1a_ref, w1d_ref, w1pn_ref, w1pp_ref, b1_ref,
               coords_ref, a1_ref, s1_ref):
    x = x_ref[0]
    a = a_ref[0, 0]
    b = b_ref[0, 0]
    c = c_ref[0, 0]
    z = a * x[:, 2:3] + b * x[:, 3:4] + c
    zeros = jnp.zeros((N, 5), jnp.float32)
    coords = jnp.concatenate([x[:, 0:2], z, zeros], axis=1)
    coords_ref[0] = coords
    f0 = jnp.maximum(jnp.dot(x, ws_ref[...],
                             preferred_element_type=jnp.float32, precision=lax.Precision.DEFAULT)
                     + bs_ref[...], 0.0)
    cp_p = jnp.dot(coords, w1pp_ref[...], preferred_element_type=jnp.float32, precision=lax.Precision.DEFAULT)
    cp_n = jnp.dot(coords, w1pn_ref[...], preferred_element_type=jnp.float32, precision=lax.Precision.DEFAULT)
    a1_ref[0] = (jnp.dot(f0, w1a_ref[...], preferred_element_type=jnp.float32, precision=lax.Precision.DEFAULT)
                 + cp_n + b1_ref[...])
    s1_ref[0] = (jnp.dot(f0, w1d_ref[...], preferred_element_type=jnp.float32, precision=lax.Precision.DEFAULT)
                 + cp_p)


def _prep_call(x, a, b, c, ws, bs, w1a, w1d, w1pn, w1pp, b1, interpret=False):
    full = lambda s: pl.BlockSpec(s, lambda i: (0,) * len(s))
    bspec = lambda s: pl.BlockSpec(s, lambda i: (i, 0, 0))
    return pl.pallas_call(
        _prep_body,
        grid=(B,),
        in_specs=[bspec((1, N, C_IN)), full((1, 1)), full((1, 1)), full((1, 1)),
                  full((C_IN, W0)), full((1, W0)),
                  full((W0, W1)), full((W0, W1)), full((8, W1)), full((8, W1)),
                  full((1, W1))],
        out_specs=[bspec((1, N, 8)), bspec((1, N, W1)), bspec((1, N, W1))],
        out_shape=[jax.ShapeDtypeStruct((B, N, 8), jnp.float32),
                   jax.ShapeDtypeStruct((B, N, W1), jnp.float32),
                   jax.ShapeDtypeStruct((B, N, W1), jnp.float32)],
        interpret=interpret,
    )(x, a, b, c, ws, bs, w1a, w1d, w1pn, w1pp, b1)


def _topk_body(ct_ref, cT_ref, idx_ref, *, prec):
    bi = pl.program_id(0)
    ct = ct_ref[0]
    cT = cT_ref[0]
    xx_t = jnp.sum(ct * ct, axis=1, keepdims=True)
    xx_f = jnp.sum(cT * cT, axis=0, keepdims=True)
    d = xx_t + xx_f - 2.0 * jnp.dot(ct, cT,
                                    preferred_element_type=jnp.float32, precision=prec)
    d = jnp.maximum(d, 0.0)
    col = lax.broadcasted_iota(jnp.int32, (M_TILE, N), 1)
    big_i = jnp.int32(1 << 30)
    inf = jnp.float32(3.0e38)
    base = bi * N
    for k in range(K):
        m = jnp.min(d, axis=1, keepdims=True)
        j = jnp.min(jnp.where(d == m, col, big_i), axis=1,
                    keepdims=True)
        idx_ref[0, :, k:k + 1] = j + base
        d = jnp.where(col == j, inf, d)


def _topk_call(coords_tiles, coords_T, interpret=False,
               prec=lax.Precision.DEFAULT):
    n_t = N // M_TILE
    return pl.pallas_call(
        functools.partial(_topk_body, prec=prec),
        grid=(B, n_t),
        in_specs=[pl.BlockSpec((1, M_TILE, 8), lambda bi, t: (bi * n_t + t, 0, 0)),
                  pl.BlockSpec((1, 8, N), lambda bi, t: (bi, 0, 0))],
        out_specs=pl.BlockSpec((1, M_TILE, K), lambda bi, t: (bi * n_t + t, 0, 0)),
        out_shape=jax.ShapeDtypeStruct((B * n_t, M_TILE, K), jnp.int32),
        interpret=interpret,
    )(coords_tiles, coords_T)


def _mid_body(a1_ref, m1_ref, coords_ref, w2a_ref, w2d_ref, w2pn_ref,
              w2pp_ref, b2_ref, a2_ref, s2_ref):
    f1 = jnp.maximum(a1_ref[0] + m1_ref[0], 0.0)
    coords = coords_ref[0]
    cp_p = jnp.dot(coords, w2pp_ref[...], preferred_element_type=jnp.float32, precision=lax.Precision.DEFAULT)
    cp_n = jnp.dot(coords, w2pn_ref[...], preferred_element_type=jnp.float32, precision=lax.Precision.DEFAULT)
    a2_ref[0] = (jnp.dot(f1, w2a_ref[...], preferred_element_type=jnp.float32, precision=lax.Precision.DEFAULT)
                 + cp_n + b2_ref[...])
    s2_ref[0] = (jnp.dot(f1, w2d_ref[...], preferred_element_type=jnp.float32, precision=lax.Precision.DEFAULT)
                 + cp_p)


def _mid_call(a1, m1, coords, w2a, w2d, w2pn, w2pp, b2, interpret=False):
    full = lambda s: pl.BlockSpec(s, lambda i: (0,) * len(s))
    bspec = lambda s: pl.BlockSpec(s, lambda i: (i, 0, 0))
    return pl.pallas_call(
        _mid_body,
        grid=(B,),
        in_specs=[bspec((1, N, W1)), bspec((1, N, W1)), bspec((1, N, 8)),
                  full((W1, W2)), full((W1, W2)), full((8, W2)), full((8, W2)),
                  full((1, W2))],
        out_specs=[bspec((1, N, W2)), bspec((1, N, W2))],
        out_shape=[jax.ShapeDtypeStruct((B, N, W2), jnp.float32),
                   jax.ShapeDtypeStruct((B, N, W2), jnp.float32)],
        interpret=interpret,
    )(a1, m1, coords, w2a, w2d, w2pn, w2pp, b2)


@functools.cache
def _make_gather_max(D):
    mesh = plsc.VectorSubcoreMesh(core_axis_name="c", subcore_axis_name="s")
    n_grp = PTS_PER_W // GRP

    @functools.partial(
        pl.kernel, mesh=mesh,
        out_type=jax.ShapeDtypeStruct((B * N, D), jnp.float32),
        scratch_types=[
            pltpu.VMEM((PTS_PER_W * K,), jnp.int32),
            pltpu.VMEM((GRP * K, D), jnp.float32),
            pltpu.VMEM((GRP * K, D), jnp.float32),
            pltpu.VMEM((GRP, D), jnp.float32),
            pltpu.SemaphoreType.DMA,
            pltpu.SemaphoreType.DMA,
        ],
    )
    def gm(s_hbm, idx_hbm, out_hbm, idx_v, r0, r1, out_v, s0, s1):
        wid = lax.axis_index("s") * NC + lax.axis_index("c")
        base = wid * PTS_PER_W
        pltpu.sync_copy(idx_hbm.at[pl.ds(base * K, PTS_PER_W * K)], idx_v)
        bufs, sems = (r0, r1), (s0, s1)

        def gather(g, buf, sem):
            ii = idx_v.at[pl.ds(g * (GRP * K), GRP * K)]
            return pltpu.make_async_copy(s_hbm.at[ii], buf, sem)

        gather(0, r0, s0).start()
        gather(1, r1, s1).start()

        def body(i, _):
            for b in range(2):
                g = 2 * i + b
                buf, sem = bufs[b], sems[b]
                gather(g, buf, sem).wait()
                for p in range(GRP):
                    for dc in range(D // 16):
                        sl = pl.ds(dc * 16, 16)
                        vals = [buf[p * K + j, sl] for j in range(K)]
                        while len(vals) > 1:
                            vals = [jnp.maximum(vals[v], vals[v + 1])
                                    for v in range(0, len(vals), 2)]
                        out_v[p, sl] = vals[0]
                pltpu.sync_copy(out_v, out_hbm.at[pl.ds(base + g * GRP, GRP)])
                gather(jnp.minimum(g + 2, n_grp - 1), buf, sem).start()
            return _

        lax.fori_loop(0, n_grp // 2, body, None)
        gather(n_grp - 1, r0, s0).wait()
        gather(n_grp - 1, r1, s1).wait()

    return gm


def _head_body(a2_ref, m2_ref, x_ref, gw_ref, gb_ref, hwf_ref, hwg_ref,
               hb_ref, h2w_ref, h2b_ref, th_ref, sh_ref, sc_ref, out_ref):
    f2 = jnp.maximum(a2_ref[0] + m2_ref[0], 0.0)
    g = jnp.max(f2, axis=0, keepdims=True)
    g = jnp.maximum(jnp.dot(g, gw_ref[...],
                            preferred_element_type=jnp.float32, precision=lax.Precision.DEFAULT)
                    + gb_ref[...], 0.0)
    gh = jnp.dot(g, hwg_ref[...], preferred_element_type=jnp.float32, precision=lax.Precision.DEFAULT) \
        + hb_ref[...]
    h = jnp.maximum(jnp.dot(f2, hwf_ref[...],
                            preferred_element_type=jnp.float32, precision=lax.Precision.DEFAULT) + gh, 0.0)
    logits = jnp.dot(h, h2w_ref[...],
                     preferred_element_type=jnp.float32, precision=lax.Precision.DEFAULT) + h2b_ref[...]
    hag = x_ref[0][:, 3:4]
    t = sh_ref[0, 0] * (th_ref[0, 0] - hag)
    bias = sc_ref[0, 0] / (1.0 + jnp.exp(-t))
    cls = lax.broadcasted_iota(jnp.int32, (N, NUM_CLASSES), 1)
    out_ref[0] = jnp.where(cls == 0, logits + bias, logits)


def _head_call(a2, m2, x, gw, gb, hwf, hwg, hb, h2w, h2b, th, sh, sc,
               interpret=False):
    full = lambda s: pl.BlockSpec(s, lambda i: (0,) * len(s))
    bspec = lambda s: pl.BlockSpec(s, lambda i: (i, 0, 0))
    return pl.pallas_call(
        _head_body,
        grid=(B,),
        in_specs=[bspec((1, N, W2)), bspec((1, N, W2)), bspec((1, N, C_IN)),
                  full((W2, W2)), full((1, W2)),
                  full((W2, W2)), full((W2, W2)), full((1, W2)),
                  full((W2, NUM_CLASSES)), full((1, NUM_CLASSES)),
                  full((1, 1)), full((1, 1)), full((1, 1))],
        out_specs=bspec((1, N, NUM_CLASSES)),
        out_shape=jax.ShapeDtypeStruct((B, N, NUM_CLASSES), jnp.float32),
        interpret=interpret,
    )(a2, m2, x, gw, gb, hwf, hwg, hb, h2w, h2b, th, sh, sc)


def kernel(x, hmix_a, hmix_b, hmix_c, stem_W, stem_b, b1_W, b1_b,
           b2_W, b2_b, glob_W, glob_b, head1_W, head1_b, head2_W, head2_b,
           hp_thresh, hp_sharp, hp_scale):
    f32 = jnp.float32
    s11 = lambda v: jnp.asarray(v, f32).reshape(1, 1)
    pad8 = lambda w: jnp.concatenate(
        [w, jnp.zeros((8 - w.shape[0], w.shape[1]), f32)], axis=0)

    w1f, w1d, w1p = b1_W[:W0], b1_W[W0:2 * W0], b1_W[2 * W0:]
    w2f, w2d, w2p = b2_W[:W1], b2_W[W1:2 * W1], b2_W[2 * W1:]

    coords, a1, s1 = _prep_call(
        x, s11(hmix_a), s11(hmix_b), s11(hmix_c),
        stem_W, stem_b.reshape(1, W0),
        w1f - w1d, w1d, pad8(-w1p), pad8(w1p), b1_b.reshape(1, W1))

    coords_T = jnp.swapaxes(coords, 1, 2)
    ct_tiles = coords.reshape(B * (N // M_TILE), M_TILE, 8)
    idx = _topk_call(ct_tiles, coords_T)
    idx_flat = idx.reshape(B * N * K)

    m1 = _make_gather_max(W1)(s1.reshape(B * N, W1), idx_flat)

    a2, s2 = _mid_call(a1, m1.reshape(B, N, W1), coords,
                       w2f - w2d, w2d, pad8(-w2p), pad8(w2p),
                       b2_b.reshape(1, W2))

    m2 = _make_gather_max(W2)(s2.reshape(B * N, W2), idx_flat)

    return _head_call(a2, m2.reshape(B, N, W2), x,
                      glob_W, glob_b.reshape(1, W2),
                      head1_W[:W2], head1_W[W2:], head1_b.reshape(1, W2),
                      head2_W, head2_b.reshape(1, NUM_CLASSES),
                      s11(hp_thresh), s11(hp_sharp), s11(hp_scale))


# speedup vs baseline: 22.3666x; 1.5503x over previous
import functools

import jax
import jax.numpy as jnp
from jax import lax
from jax.experimental import pallas as pl
from jax.experimental.pallas import tpu as pltpu
from jax.experimental.pallas import tpu_sc as plsc

B, N, C_IN = 4, 4096, 4
K = 16
W0, W1, W2 = 64, 128, 256
NUM_CLASSES = 3

NC, NS = 2, 16
NW = NC * NS
PTS_PER_W = (B * N) // NW
GRP = 8
M_TILE = 256


def _prep_body(x_ref, a_ref, b_ref, c_ref, ws_ref, bs_ref,
               w1a_ref, w1d_ref, w1pn_ref, w1pp_ref, b1_ref,
               coords_ref, a1_ref, s1_ref):
    x = x_ref[0]
    a = a_ref[0, 0]
    b = b_ref[0, 0]
    c = c_ref[0, 0]
    z = a * x[:, 2:3] + b * x[:, 3:4] + c
    zeros = jnp.zeros((N, 5), jnp.float32)
    coords = jnp.concatenate([x[:, 0:2], z, zeros], axis=1)
    coords_ref[0] = coords
    f0 = jnp.maximum(jnp.dot(x, ws_ref[...],
                             preferred_element_type=jnp.float32, precision=lax.Precision.DEFAULT)
                     + bs_ref[...], 0.0)
    cp_p = jnp.dot(coords, w1pp_ref[...], preferred_element_type=jnp.float32, precision=lax.Precision.DEFAULT)
    cp_n = jnp.dot(coords, w1pn_ref[...], preferred_element_type=jnp.float32, precision=lax.Precision.DEFAULT)
    a1_ref[0] = (jnp.dot(f0, w1a_ref[...], preferred_element_type=jnp.float32, precision=lax.Precision.DEFAULT)
                 + cp_n + b1_ref[...])
    s1_ref[0] = (jnp.dot(f0, w1d_ref[...], preferred_element_type=jnp.float32, precision=lax.Precision.DEFAULT)
                 + cp_p)


def _prep_call(x, a, b, c, ws, bs, w1a, w1d, w1pn, w1pp, b1, interpret=False):
    full = lambda s: pl.BlockSpec(s, lambda i: (0,) * len(s))
    bspec = lambda s: pl.BlockSpec(s, lambda i: (i, 0, 0))
    return pl.pallas_call(
        _prep_body,
        grid=(B,),
        in_specs=[bspec((1, N, C_IN)), full((1, 1)), full((1, 1)), full((1, 1)),
                  full((C_IN, W0)), full((1, W0)),
                  full((W0, W1)), full((W0, W1)), full((8, W1)), full((8, W1)),
                  full((1, W1))],
        out_specs=[bspec((1, N, 8)), bspec((1, N, W1)), bspec((1, N, W1))],
        out_shape=[jax.ShapeDtypeStruct((B, N, 8), jnp.float32),
                   jax.ShapeDtypeStruct((B, N, W1), jnp.float32),
                   jax.ShapeDtypeStruct((B, N, W1), jnp.float32)],
        interpret=interpret,
    )(x, a, b, c, ws, bs, w1a, w1d, w1pn, w1pp, b1)


def _topk_body(ct_ref, cT_ref, idx_ref, *, prec):
    bi = pl.program_id(0)
    ct = ct_ref[0]
    cT = cT_ref[0]
    xx_t = jnp.sum(ct * ct, axis=1, keepdims=True)
    xx_f = jnp.sum(cT * cT, axis=0, keepdims=True)
    d = xx_t + xx_f - 2.0 * jnp.dot(ct, cT,
                                    preferred_element_type=jnp.float32, precision=prec)
    d = jnp.maximum(d, 0.0)
    NL = 128
    inf = jnp.float32(3.0e38)
    bigf = jnp.float32(1.0e9)
    lane = lax.broadcasted_iota(jnp.int32, (M_TILE, NL), 1).astype(jnp.float32)
    v = [jnp.full((M_TILE, NL), inf, jnp.float32) for _ in range(4)]
    iv = [jnp.zeros((M_TILE, NL), jnp.float32) for _ in range(4)]
    for s in range(N // NL):
        x = lax.slice(d, (0, s * NL), (M_TILE, (s + 1) * NL))
        ix = lane + jnp.float32(s * NL)
        g0, g1 = x < v[0], x < v[1]
        g2, g3 = x < v[2], x < v[3]
        v = [jnp.where(g0, x, v[0]),
             jnp.where(g1, jnp.where(g0, v[0], x), v[1]),
             jnp.where(g2, jnp.where(g1, v[1], x), v[2]),
             jnp.where(g3, jnp.where(g2, v[2], x), v[3])]
        iv = [jnp.where(g0, ix, iv[0]),
              jnp.where(g1, jnp.where(g0, iv[0], ix), iv[1]),
              jnp.where(g2, jnp.where(g1, iv[1], ix), iv[2]),
              jnp.where(g3, jnp.where(g2, iv[2], ix), iv[3])]
    base = bi * N
    for k in range(K):
        m = jnp.min(v[0], axis=1, keepdims=True)
        jf = jnp.min(jnp.where(v[0] == m, iv[0], bigf), axis=1,
                     keepdims=True)
        idx_ref[0, :, k:k + 1] = jf.astype(jnp.int32) + base
        eq = iv[0] == jf
        v = [jnp.where(eq, v[1], v[0]), jnp.where(eq, v[2], v[1]),
             jnp.where(eq, v[3], v[2]), jnp.where(eq, inf, v[3])]
        iv = [jnp.where(eq, iv[1], iv[0]), jnp.where(eq, iv[2], iv[1]),
              jnp.where(eq, iv[3], iv[2]), iv[3]]


def _topk_call(coords_tiles, coords_T, interpret=False,
               prec=lax.Precision.DEFAULT):
    n_t = N // M_TILE
    return pl.pallas_call(
        functools.partial(_topk_body, prec=prec),
        grid=(B, n_t),
        in_specs=[pl.BlockSpec((1, M_TILE, 8), lambda bi, t: (bi * n_t + t, 0, 0)),
                  pl.BlockSpec((1, 8, N), lambda bi, t: (bi, 0, 0))],
        out_specs=pl.BlockSpec((1, M_TILE, K), lambda bi, t: (bi * n_t + t, 0, 0)),
        out_shape=jax.ShapeDtypeStruct((B * n_t, M_TILE, K), jnp.int32),
        interpret=interpret,
    )(coords_tiles, coords_T)


def _mid_body(a1_ref, m1_ref, coords_ref, w2a_ref, w2d_ref, w2pn_ref,
              w2pp_ref, b2_ref, a2_ref, s2_ref):
    f1 = jnp.maximum(a1_ref[0] + m1_ref[0], 0.0)
    coords = coords_ref[0]
    cp_p = jnp.dot(coords, w2pp_ref[...], preferred_element_type=jnp.float32, precision=lax.Precision.DEFAULT)
    cp_n = jnp.dot(coords, w2pn_ref[...], preferred_element_type=jnp.float32, precision=lax.Precision.DEFAULT)
    a2_ref[0] = (jnp.dot(f1, w2a_ref[...], preferred_element_type=jnp.float32, precision=lax.Precision.DEFAULT)
                 + cp_n + b2_ref[...])
    s2_ref[0] = (jnp.dot(f1, w2d_ref[...], preferred_element_type=jnp.float32, precision=lax.Precision.DEFAULT)
                 + cp_p)


def _mid_call(a1, m1, coords, w2a, w2d, w2pn, w2pp, b2, interpret=False):
    full = lambda s: pl.BlockSpec(s, lambda i: (0,) * len(s))
    bspec = lambda s: pl.BlockSpec(s, lambda i: (i, 0, 0))
    return pl.pallas_call(
        _mid_body,
        grid=(B,),
        in_specs=[bspec((1, N, W1)), bspec((1, N, W1)), bspec((1, N, 8)),
                  full((W1, W2)), full((W1, W2)), full((8, W2)), full((8, W2)),
                  full((1, W2))],
        out_specs=[bspec((1, N, W2)), bspec((1, N, W2))],
        out_shape=[jax.ShapeDtypeStruct((B, N, W2), jnp.float32),
                   jax.ShapeDtypeStruct((B, N, W2), jnp.float32)],
        interpret=interpret,
    )(a1, m1, coords, w2a, w2d, w2pn, w2pp, b2)


@functools.cache
def _make_gather_max(D):
    mesh = plsc.VectorSubcoreMesh(core_axis_name="c", subcore_axis_name="s")
    n_grp = PTS_PER_W // GRP

    @functools.partial(
        pl.kernel, mesh=mesh,
        out_type=jax.ShapeDtypeStruct((B * N, D), jnp.float32),
        scratch_types=[
            pltpu.VMEM((PTS_PER_W * K,), jnp.int32),
            pltpu.VMEM((GRP * K, D), jnp.float32),
            pltpu.VMEM((GRP * K, D), jnp.float32),
            pltpu.VMEM((GRP, D), jnp.float32),
            pltpu.SemaphoreType.DMA,
            pltpu.SemaphoreType.DMA,
        ],
    )
    def gm(s_hbm, idx_hbm, out_hbm, idx_v, r0, r1, out_v, s0, s1):
        wid = lax.axis_index("s") * NC + lax.axis_index("c")
        base = wid * PTS_PER_W
        pltpu.sync_copy(idx_hbm.at[pl.ds(base * K, PTS_PER_W * K)], idx_v)
        bufs, sems = (r0, r1), (s0, s1)

        def gather(g, buf, sem):
            ii = idx_v.at[pl.ds(g * (GRP * K), GRP * K)]
            return pltpu.make_async_copy(s_hbm.at[ii], buf, sem)

        gather(0, r0, s0).start()
        gather(1, r1, s1).start()

        def body(i, _):
            for b in range(2):
                g = 2 * i + b
                buf, sem = bufs[b], sems[b]
                gather(g, buf, sem).wait()
                for p in range(GRP):
                    for dc in range(D // 16):
                        sl = pl.ds(dc * 16, 16)
                        vals = [buf[p * K + j, sl] for j in range(K)]
                        while len(vals) > 1:
                            vals = [jnp.maximum(vals[v], vals[v + 1])
                                    for v in range(0, len(vals), 2)]
                        out_v[p, sl] = vals[0]
                pltpu.sync_copy(out_v, out_hbm.at[pl.ds(base + g * GRP, GRP)])
                gather(jnp.minimum(g + 2, n_grp - 1), buf, sem).start()
            return _

        lax.fori_loop(0, n_grp // 2, body, None)
        gather(n_grp - 1, r0, s0).wait()
        gather(n_grp - 1, r1, s1).wait()

    return gm


def _head_body(a2_ref, m2_ref, x_ref, gw_ref, gb_ref, hwf_ref, hwg_ref,
               hb_ref, h2w_ref, h2b_ref, th_ref, sh_ref, sc_ref, out_ref):
    f2 = jnp.maximum(a2_ref[0] + m2_ref[0], 0.0)
    g = jnp.max(f2, axis=0, keepdims=True)
    g = jnp.maximum(jnp.dot(g, gw_ref[...],
                            preferred_element_type=jnp.float32, precision=lax.Precision.DEFAULT)
                    + gb_ref[...], 0.0)
    gh = jnp.dot(g, hwg_ref[...], preferred_element_type=jnp.float32, precision=lax.Precision.DEFAULT) \
        + hb_ref[...]
    h = jnp.maximum(jnp.dot(f2, hwf_ref[...],
                            preferred_element_type=jnp.float32, precision=lax.Precision.DEFAULT) + gh, 0.0)
    logits = jnp.dot(h, h2w_ref[...],
                     preferred_element_type=jnp.float32, precision=lax.Precision.DEFAULT) + h2b_ref[...]
    hag = x_ref[0][:, 3:4]
    t = sh_ref[0, 0] * (th_ref[0, 0] - hag)
    bias = sc_ref[0, 0] / (1.0 + jnp.exp(-t))
    cls = lax.broadcasted_iota(jnp.int32, (N, NUM_CLASSES), 1)
    out_ref[0] = jnp.where(cls == 0, logits + bias, logits)


def _head_call(a2, m2, x, gw, gb, hwf, hwg, hb, h2w, h2b, th, sh, sc,
               interpret=False):
    full = lambda s: pl.BlockSpec(s, lambda i: (0,) * len(s))
    bspec = lambda s: pl.BlockSpec(s, lambda i: (i, 0, 0))
    return pl.pallas_call(
        _head_body,
        grid=(B,),
        in_specs=[bspec((1, N, W2)), bspec((1, N, W2)), bspec((1, N, C_IN)),
                  full((W2, W2)), full((1, W2)),
                  full((W2, W2)), full((W2, W2)), full((1, W2)),
                  full((W2, NUM_CLASSES)), full((1, NUM_CLASSES)),
                  full((1, 1)), full((1, 1)), full((1, 1))],
        out_specs=bspec((1, N, NUM_CLASSES)),
        out_shape=jax.ShapeDtypeStruct((B, N, NUM_CLASSES), jnp.float32),
        interpret=interpret,
    )(a2, m2, x, gw, gb, hwf, hwg, hb, h2w, h2b, th, sh, sc)


def kernel(x, hmix_a, hmix_b, hmix_c, stem_W, stem_b, b1_W, b1_b,
           b2_W, b2_b, glob_W, glob_b, head1_W, head1_b, head2_W, head2_b,
           hp_thresh, hp_sharp, hp_scale):
    f32 = jnp.float32
    s11 = lambda v: jnp.asarray(v, f32).reshape(1, 1)
    pad8 = lambda w: jnp.concatenate(
        [w, jnp.zeros((8 - w.shape[0], w.shape[1]), f32)], axis=0)

    w1f, w1d, w1p = b1_W[:W0], b1_W[W0:2 * W0], b1_W[2 * W0:]
    w2f, w2d, w2p = b2_W[:W1], b2_W[W1:2 * W1], b2_W[2 * W1:]

    coords, a1, s1 = _prep_call(
        x, s11(hmix_a), s11(hmix_b), s11(hmix_c),
        stem_W, stem_b.reshape(1, W0),
        w1f - w1d, w1d, pad8(-w1p), pad8(w1p), b1_b.reshape(1, W1))

    coords_T = jnp.swapaxes(coords, 1, 2)
    ct_tiles = coords.reshape(B * (N // M_TILE), M_TILE, 8)
    idx = _topk_call(ct_tiles, coords_T)
    idx_flat = idx.reshape(B * N * K)

    m1 = _make_gather_max(W1)(s1.reshape(B * N, W1), idx_flat)

    a2, s2 = _mid_call(a1, m1.reshape(B, N, W1), coords,
                       w2f - w2d, w2d, pad8(-w2p), pad8(w2p),
                       b2_b.reshape(1, W2))

    m2 = _make_gather_max(W2)(s2.reshape(B * N, W2), idx_flat)

    return _head_call(a2, m2.reshape(B, N, W2), x,
                      glob_W, glob_b.reshape(1, W2),
                      head1_W[:W2], head1_W[W2:], head1_b.reshape(1, W2),
                      head2_W, head2_b.reshape(1, NUM_CLASSES),
                      s11(hp_thresh), s11(hp_sharp), s11(hp_scale))


# speedup vs baseline: 22.5214x; 1.0069x over previous
import functools

import jax
import jax.numpy as jnp
from jax import lax
from jax.experimental import pallas as pl
from jax.experimental.pallas import tpu as pltpu
from jax.experimental.pallas import tpu_sc as plsc

B, N, C_IN = 4, 4096, 4
K = 16
W0, W1, W2 = 64, 128, 256
NUM_CLASSES = 3

NC, NS = 2, 16
NW = NC * NS
PTS_PER_W = (B * N) // NW
GRP = 8
M_TILE = 256


def _prep_body(x_ref, a_ref, b_ref, c_ref, ws_ref, bs_ref,
               w1a_ref, w1d_ref, w1pn_ref, w1pp_ref, b1_ref,
               coords_ref, a1_ref, s1_ref):
    x = x_ref[0]
    a = a_ref[0, 0]
    b = b_ref[0, 0]
    c = c_ref[0, 0]
    z = a * x[:, 2:3] + b * x[:, 3:4] + c
    zeros = jnp.zeros((N, 5), jnp.float32)
    coords = jnp.concatenate([x[:, 0:2], z, zeros], axis=1)
    coords_ref[0] = coords
    f0 = jnp.maximum(jnp.dot(x, ws_ref[...],
                             preferred_element_type=jnp.float32, precision=lax.Precision.DEFAULT)
                     + bs_ref[...], 0.0)
    cp_p = jnp.dot(coords, w1pp_ref[...], preferred_element_type=jnp.float32, precision=lax.Precision.DEFAULT)
    cp_n = jnp.dot(coords, w1pn_ref[...], preferred_element_type=jnp.float32, precision=lax.Precision.DEFAULT)
    a1_ref[0] = (jnp.dot(f0, w1a_ref[...], preferred_element_type=jnp.float32, precision=lax.Precision.DEFAULT)
                 + cp_n + b1_ref[...])
    s1_ref[0] = (jnp.dot(f0, w1d_ref[...], preferred_element_type=jnp.float32, precision=lax.Precision.DEFAULT)
                 + cp_p)


def _prep_call(x, a, b, c, ws, bs, w1a, w1d, w1pn, w1pp, b1, interpret=False):
    full = lambda s: pl.BlockSpec(s, lambda i: (0,) * len(s))
    bspec = lambda s: pl.BlockSpec(s, lambda i: (i, 0, 0))
    return pl.pallas_call(
        _prep_body,
        grid=(B,),
        in_specs=[bspec((1, N, C_IN)), full((1, 1)), full((1, 1)), full((1, 1)),
                  full((C_IN, W0)), full((1, W0)),
                  full((W0, W1)), full((W0, W1)), full((8, W1)), full((8, W1)),
                  full((1, W1))],
        out_specs=[bspec((1, N, 8)), bspec((1, N, W1)), bspec((1, N, W1))],
        out_shape=[jax.ShapeDtypeStruct((B, N, 8), jnp.float32),
                   jax.ShapeDtypeStruct((B, N, W1), jnp.float32),
                   jax.ShapeDtypeStruct((B, N, W1), jnp.float32)],
        interpret=interpret,
    )(x, a, b, c, ws, bs, w1a, w1d, w1pn, w1pp, b1)


def _topk_body(ct_ref, cT_ref, idx_ref, *, prec):
    bi = pl.program_id(0)
    ct = ct_ref[0]
    cT = cT_ref[0]
    xx_t = jnp.sum(ct * ct, axis=1, keepdims=True)
    xx_f = jnp.sum(cT * cT, axis=0, keepdims=True)
    d = xx_t + xx_f - 2.0 * jnp.dot(ct, cT,
                                    preferred_element_type=jnp.float32, precision=prec)
    d = jnp.maximum(d, 0.0)
    NL = 128
    inf = jnp.float32(3.0e38)
    bigf = jnp.float32(1.0e9)
    lane = lax.broadcasted_iota(jnp.int32, (M_TILE, NL), 1).astype(jnp.float32)
    v = [jnp.full((M_TILE, NL), inf, jnp.float32) for _ in range(4)]
    iv = [jnp.zeros((M_TILE, NL), jnp.float32) for _ in range(4)]
    for s in range(N // NL):
        x = lax.slice(d, (0, s * NL), (M_TILE, (s + 1) * NL))
        ix = lane + jnp.float32(s * NL)
        g0, g1 = x < v[0], x < v[1]
        g2, g3 = x < v[2], x < v[3]
        v = [jnp.where(g0, x, v[0]),
             jnp.where(g1, jnp.where(g0, v[0], x), v[1]),
             jnp.where(g2, jnp.where(g1, v[1], x), v[2]),
             jnp.where(g3, jnp.where(g2, v[2], x), v[3])]
        iv = [jnp.where(g0, ix, iv[0]),
              jnp.where(g1, jnp.where(g0, iv[0], ix), iv[1]),
              jnp.where(g2, jnp.where(g1, iv[1], ix), iv[2]),
              jnp.where(g3, jnp.where(g2, iv[2], ix), iv[3])]
    base = bi * N
    for k in range(K):
        m = jnp.min(v[0], axis=1, keepdims=True)
        jf = jnp.min(jnp.where(v[0] == m, iv[0], bigf), axis=1,
                     keepdims=True)
        idx_ref[0, :, k:k + 1] = jf.astype(jnp.int32) + base
        eq = iv[0] == jf
        v = [jnp.where(eq, v[1], v[0]), jnp.where(eq, v[2], v[1]),
             jnp.where(eq, v[3], v[2]), jnp.where(eq, inf, v[3])]
        iv = [jnp.where(eq, iv[1], iv[0]), jnp.where(eq, iv[2], iv[1]),
              jnp.where(eq, iv[3], iv[2]), iv[3]]


def _topk_call(coords_tiles, coords_T, interpret=False,
               prec=lax.Precision.DEFAULT):
    n_t = N // M_TILE
    return pl.pallas_call(
        functools.partial(_topk_body, prec=prec),
        grid=(B, n_t),
        in_specs=[pl.BlockSpec((1, M_TILE, 8), lambda bi, t: (bi * n_t + t, 0, 0)),
                  pl.BlockSpec((1, 8, N), lambda bi, t: (bi, 0, 0))],
        out_specs=pl.BlockSpec((1, M_TILE, K), lambda bi, t: (bi * n_t + t, 0, 0)),
        out_shape=jax.ShapeDtypeStruct((B * n_t, M_TILE, K), jnp.int32),
        interpret=interpret,
    )(coords_tiles, coords_T)


def _mid_body(a1_ref, m1_ref, coords_ref, w2a_ref, w2d_ref, w2pn_ref,
              w2pp_ref, b2_ref, a2_ref, s2_ref):
    f1 = jnp.maximum(a1_ref[0] + m1_ref[0], 0.0)
    coords = coords_ref[0]
    cp_p = jnp.dot(coords, w2pp_ref[...], preferred_element_type=jnp.float32, precision=lax.Precision.DEFAULT)
    cp_n = jnp.dot(coords, w2pn_ref[...], preferred_element_type=jnp.float32, precision=lax.Precision.DEFAULT)
    a2_ref[0] = (jnp.dot(f1, w2a_ref[...], preferred_element_type=jnp.float32, precision=lax.Precision.DEFAULT)
                 + cp_n + b2_ref[...])
    s2_ref[0] = (jnp.dot(f1, w2d_ref[...], preferred_element_type=jnp.float32, precision=lax.Precision.DEFAULT)
                 + cp_p)


def _mid_call(a1, m1, coords, w2a, w2d, w2pn, w2pp, b2, interpret=False):
    full = lambda s: pl.BlockSpec(s, lambda i: (0,) * len(s))
    bspec = lambda s: pl.BlockSpec(s, lambda i: (i, 0, 0))
    return pl.pallas_call(
        _mid_body,
        grid=(B,),
        in_specs=[bspec((1, N, W1)), bspec((1, N, W1)), bspec((1, N, 8)),
                  full((W1, W2)), full((W1, W2)), full((8, W2)), full((8, W2)),
                  full((1, W2))],
        out_specs=[bspec((1, N, W2)), bspec((1, N, W2))],
        out_shape=[jax.ShapeDtypeStruct((B, N, W2), jnp.float32),
                   jax.ShapeDtypeStruct((B, N, W2), jnp.float32)],
        interpret=interpret,
    )(a1, m1, coords, w2a, w2d, w2pn, w2pp, b2)


@functools.cache
def _make_gather_max(D):
    mesh = plsc.VectorSubcoreMesh(core_axis_name="c", subcore_axis_name="s")
    n_grp = PTS_PER_W // GRP

    @functools.partial(
        pl.kernel, mesh=mesh,
        out_type=jax.ShapeDtypeStruct((B * N, D), jnp.float32),
        scratch_types=[
            pltpu.VMEM((PTS_PER_W * K,), jnp.int32),
            pltpu.VMEM((GRP * K, D), jnp.float32),
            pltpu.VMEM((GRP * K, D), jnp.float32),
            pltpu.VMEM((GRP, D), jnp.float32),
            pltpu.VMEM((GRP, D), jnp.float32),
            pltpu.SemaphoreType.DMA,
            pltpu.SemaphoreType.DMA,
            pltpu.SemaphoreType.DMA,
            pltpu.SemaphoreType.DMA,
        ],
    )
    def gm(s_hbm, idx_hbm, out_hbm, idx_v, r0, r1, o0, o1, s0, s1, t0, t1):
        wid = lax.axis_index("s") * NC + lax.axis_index("c")
        base = wid * PTS_PER_W
        pltpu.sync_copy(idx_hbm.at[pl.ds(base * K, PTS_PER_W * K)], idx_v)
        bufs, sems = (r0, r1), (s0, s1)
        obufs, osems = (o0, o1), (t0, t1)

        def gather(g, buf, sem):
            ii = idx_v.at[pl.ds(g * (GRP * K), GRP * K)]
            return pltpu.make_async_copy(s_hbm.at[ii], buf, sem)

        def put(g, obuf, osem):
            return pltpu.make_async_copy(
                obuf, out_hbm.at[pl.ds(base + g * GRP, GRP)], osem)

        gather(0, r0, s0).start()
        gather(1, r1, s1).start()

        def body(i, _):
            for b in range(2):
                g = 2 * i + b
                buf, sem = bufs[b], sems[b]
                obuf, osem = obufs[b], osems[b]
                gather(g, buf, sem).wait()

                @pl.when(i > 0)
                def _w():
                    put(g, obuf, osem).wait()

                for p in range(GRP):
                    for dc in range(D // 16):
                        sl = pl.ds(dc * 16, 16)
                        vals = [buf[p * K + j, sl] for j in range(K)]
                        while len(vals) > 1:
                            vals = [jnp.maximum(vals[v], vals[v + 1])
                                    for v in range(0, len(vals), 2)]
                        obuf[p, sl] = vals[0]
                put(g, obuf, osem).start()
                gather(jnp.minimum(g + 2, n_grp - 1), buf, sem).start()
            return _

        lax.fori_loop(0, n_grp // 2, body, None)
        gather(n_grp - 1, r0, s0).wait()
        gather(n_grp - 1, r1, s1).wait()
        put(n_grp - 2, o0, t0).wait()
        put(n_grp - 1, o1, t1).wait()

    return gm


def _head_body(a2_ref, m2_ref, x_ref, gw_ref, gb_ref, hwf_ref, hwg_ref,
               hb_ref, h2w_ref, h2b_ref, th_ref, sh_ref, sc_ref, out_ref):
    f2 = jnp.maximum(a2_ref[0] + m2_ref[0], 0.0)
    g = jnp.max(f2, axis=0, keepdims=True)
    g = jnp.maximum(jnp.dot(g, gw_ref[...],
                            preferred_element_type=jnp.float32, precision=lax.Precision.DEFAULT)
                    + gb_ref[...], 0.0)
    gh = jnp.dot(g, hwg_ref[...], preferred_element_type=jnp.float32, precision=lax.Precision.DEFAULT) \
        + hb_ref[...]
    h = jnp.maximum(jnp.dot(f2, hwf_ref[...],
                            preferred_element_type=jnp.float32, precision=lax.Precision.DEFAULT) + gh, 0.0)
    logits = jnp.dot(h, h2w_ref[...],
                     preferred_element_type=jnp.float32, precision=lax.Precision.DEFAULT) + h2b_ref[...]
    hag = x_ref[0][:, 3:4]
    t = sh_ref[0, 0] * (th_ref[0, 0] - hag)
    bias = sc_ref[0, 0] / (1.0 + jnp.exp(-t))
    cls = lax.broadcasted_iota(jnp.int32, (N, NUM_CLASSES), 1)
    out_ref[0] = jnp.where(cls == 0, logits + bias, logits)


def _head_call(a2, m2, x, gw, gb, hwf, hwg, hb, h2w, h2b, th, sh, sc,
               interpret=False):
    full = lambda s: pl.BlockSpec(s, lambda i: (0,) * len(s))
    bspec = lambda s: pl.BlockSpec(s, lambda i: (i, 0, 0))
    return pl.pallas_call(
        _head_body,
        grid=(B,),
        in_specs=[bspec((1, N, W2)), bspec((1, N, W2)), bspec((1, N, C_IN)),
                  full((W2, W2)), full((1, W2)),
                  full((W2, W2)), full((W2, W2)), full((1, W2)),
                  full((W2, NUM_CLASSES)), full((1, NUM_CLASSES)),
                  full((1, 1)), full((1, 1)), full((1, 1))],
        out_specs=bspec((1, N, NUM_CLASSES)),
        out_shape=jax.ShapeDtypeStruct((B, N, NUM_CLASSES), jnp.float32),
        interpret=interpret,
    )(a2, m2, x, gw, gb, hwf, hwg, hb, h2w, h2b, th, sh, sc)


def kernel(x, hmix_a, hmix_b, hmix_c, stem_W, stem_b, b1_W, b1_b,
           b2_W, b2_b, glob_W, glob_b, head1_W, head1_b, head2_W, head2_b,
           hp_thresh, hp_sharp, hp_scale):
    f32 = jnp.float32
    s11 = lambda v: jnp.asarray(v, f32).reshape(1, 1)
    pad8 = lambda w: jnp.concatenate(
        [w, jnp.zeros((8 - w.shape[0], w.shape[1]), f32)], axis=0)

    w1f, w1d, w1p = b1_W[:W0], b1_W[W0:2 * W0], b1_W[2 * W0:]
    w2f, w2d, w2p = b2_W[:W1], b2_W[W1:2 * W1], b2_W[2 * W1:]

    coords, a1, s1 = _prep_call(
        x, s11(hmix_a), s11(hmix_b), s11(hmix_c),
        stem_W, stem_b.reshape(1, W0),
        w1f - w1d, w1d, pad8(-w1p), pad8(w1p), b1_b.reshape(1, W1))

    coords_T = jnp.swapaxes(coords, 1, 2)
    ct_tiles = coords.reshape(B * (N // M_TILE), M_TILE, 8)
    idx = _topk_call(ct_tiles, coords_T)
    idx_flat = idx.reshape(B * N * K)

    m1 = _make_gather_max(W1)(s1.reshape(B * N, W1), idx_flat)

    a2, s2 = _mid_call(a1, m1.reshape(B, N, W1), coords,
                       w2f - w2d, w2d, pad8(-w2p), pad8(w2p),
                       b2_b.reshape(1, W2))

    m2 = _make_gather_max(W2)(s2.reshape(B * N, W2), idx_flat)

    return _head_call(a2, m2.reshape(B, N, W2), x,
                      glob_W, glob_b.reshape(1, W2),
                      head1_W[:W2], head1_W[W2:], head1_b.reshape(1, W2),
                      head2_W, head2_b.reshape(1, NUM_CLASSES),
                      s11(hp_thresh), s11(hp_sharp), s11(hp_scale))


# speedup vs baseline: 39.4218x; 1.7504x over previous
import functools

import jax
import jax.numpy as jnp
from jax import lax
from jax.experimental import pallas as pl
from jax.experimental.pallas import tpu as pltpu
from jax.experimental.pallas import tpu_sc as plsc

B, N, C_IN = 4, 4096, 4
K = 16
W0, W1, W2 = 64, 128, 256
NUM_CLASSES = 3

NC, NS = 2, 16
NW = NC * NS
PTS_PER_W = (B * N) // NW
GRP = 8
M_TILE = 256


def _prep_body(x_ref, a_ref, b_ref, c_ref, ws_ref, bs_ref,
               w1a_ref, w1d_ref, w1pn_ref, w1pp_ref, b1_ref,
               coords_ref, a1_ref, s1_ref):
    x = x_ref[0]
    a = a_ref[0, 0]
    b = b_ref[0, 0]
    c = c_ref[0, 0]
    z = a * x[:, 2:3] + b * x[:, 3:4] + c
    zeros = jnp.zeros((N, 5), jnp.float32)
    coords = jnp.concatenate([x[:, 0:2], z, zeros], axis=1)
    coords_ref[0] = coords
    f0 = jnp.maximum(jnp.dot(x, ws_ref[...],
                             preferred_element_type=jnp.float32, precision=lax.Precision.DEFAULT)
                     + bs_ref[...], 0.0)
    cp_p = jnp.dot(coords, w1pp_ref[...], preferred_element_type=jnp.float32, precision=lax.Precision.DEFAULT)
    cp_n = jnp.dot(coords, w1pn_ref[...], preferred_element_type=jnp.float32, precision=lax.Precision.DEFAULT)
    a1_ref[0] = (jnp.dot(f0, w1a_ref[...], preferred_element_type=jnp.float32, precision=lax.Precision.DEFAULT)
                 + cp_n + b1_ref[...])
    s1_ref[0] = (jnp.dot(f0, w1d_ref[...], preferred_element_type=jnp.float32, precision=lax.Precision.DEFAULT)
                 + cp_p)


def _prep_call(x, a, b, c, ws, bs, w1a, w1d, w1pn, w1pp, b1, interpret=False):
    full = lambda s: pl.BlockSpec(s, lambda i: (0,) * len(s))
    bspec = lambda s: pl.BlockSpec(s, lambda i: (i, 0, 0))
    return pl.pallas_call(
        _prep_body,
        grid=(B,),
        in_specs=[bspec((1, N, C_IN)), full((1, 1)), full((1, 1)), full((1, 1)),
                  full((C_IN, W0)), full((1, W0)),
                  full((W0, W1)), full((W0, W1)), full((8, W1)), full((8, W1)),
                  full((1, W1))],
        out_specs=[bspec((1, N, 8)), bspec((1, N, W1)), bspec((1, N, W1))],
        out_shape=[jax.ShapeDtypeStruct((B, N, 8), jnp.float32),
                   jax.ShapeDtypeStruct((B, N, W1), jnp.float32),
                   jax.ShapeDtypeStruct((B, N, W1), jnp.float32)],
        interpret=interpret,
    )(x, a, b, c, ws, bs, w1a, w1d, w1pn, w1pp, b1)


def _topk_body(ct_ref, cT_ref, idx_ref, *, prec):
    bi = pl.program_id(0)
    ct = ct_ref[0]
    cT = cT_ref[0]
    xx_t = jnp.sum(ct * ct, axis=1, keepdims=True)
    xx_f = jnp.sum(cT * cT, axis=0, keepdims=True)
    d = xx_t + xx_f - 2.0 * jnp.dot(ct, cT,
                                    preferred_element_type=jnp.float32, precision=prec)
    d = jnp.maximum(d, 0.0)
    NL = 128
    inf = jnp.float32(3.0e38)
    bigf = jnp.float32(1.0e9)
    lane = lax.broadcasted_iota(jnp.int32, (M_TILE, NL), 1).astype(jnp.float32)
    v = [jnp.full((M_TILE, NL), inf, jnp.float32) for _ in range(4)]
    iv = [jnp.zeros((M_TILE, NL), jnp.float32) for _ in range(4)]
    for s in range(N // NL):
        x = lax.slice(d, (0, s * NL), (M_TILE, (s + 1) * NL))
        ix = lane + jnp.float32(s * NL)
        g0, g1 = x < v[0], x < v[1]
        g2, g3 = x < v[2], x < v[3]
        v = [jnp.where(g0, x, v[0]),
             jnp.where(g1, jnp.where(g0, v[0], x), v[1]),
             jnp.where(g2, jnp.where(g1, v[1], x), v[2]),
             jnp.where(g3, jnp.where(g2, v[2], x), v[3])]
        iv = [jnp.where(g0, ix, iv[0]),
              jnp.where(g1, jnp.where(g0, iv[0], ix), iv[1]),
              jnp.where(g2, jnp.where(g1, iv[1], ix), iv[2]),
              jnp.where(g3, jnp.where(g2, iv[2], ix), iv[3])]
    base = bi * N
    for k in range(K):
        m = jnp.min(v[0], axis=1, keepdims=True)
        jf = jnp.min(jnp.where(v[0] == m, iv[0], bigf), axis=1,
                     keepdims=True)
        idx_ref[0, :, k:k + 1] = jf.astype(jnp.int32) + base
        eq = iv[0] == jf
        v = [jnp.where(eq, v[1], v[0]), jnp.where(eq, v[2], v[1]),
             jnp.where(eq, v[3], v[2]), jnp.where(eq, inf, v[3])]
        iv = [jnp.where(eq, iv[1], iv[0]), jnp.where(eq, iv[2], iv[1]),
              jnp.where(eq, iv[3], iv[2]), iv[3]]


def _topk_call(coords_tiles, coords_T, interpret=False,
               prec=lax.Precision.DEFAULT):
    n_t = N // M_TILE
    return pl.pallas_call(
        functools.partial(_topk_body, prec=prec),
        grid=(B, n_t),
        in_specs=[pl.BlockSpec((1, M_TILE, 8), lambda bi, t: (bi * n_t + t, 0, 0)),
                  pl.BlockSpec((1, 8, N), lambda bi, t: (bi, 0, 0))],
        out_specs=pl.BlockSpec((1, M_TILE, K), lambda bi, t: (bi * n_t + t, 0, 0)),
        out_shape=jax.ShapeDtypeStruct((B * n_t, M_TILE, K), jnp.int32),
        interpret=interpret,
    )(coords_tiles, coords_T)


def _mid_body(a1_ref, m1_ref, coords_ref, w2a_ref, w2d_ref, w2pn_ref,
              w2pp_ref, b2_ref, a2_ref, s2_ref):
    f1 = jnp.maximum(a1_ref[0] + m1_ref[0], 0.0)
    coords = coords_ref[0]
    cp_p = jnp.dot(coords, w2pp_ref[...], preferred_element_type=jnp.float32, precision=lax.Precision.DEFAULT)
    cp_n = jnp.dot(coords, w2pn_ref[...], preferred_element_type=jnp.float32, precision=lax.Precision.DEFAULT)
    a2_ref[0] = (jnp.dot(f1, w2a_ref[...], preferred_element_type=jnp.float32, precision=lax.Precision.DEFAULT)
                 + cp_n + b2_ref[...])
    s2_ref[0] = (jnp.dot(f1, w2d_ref[...], preferred_element_type=jnp.float32, precision=lax.Precision.DEFAULT)
                 + cp_p)


def _mid_call(a1, m1, coords, w2a, w2d, w2pn, w2pp, b2, interpret=False):
    full = lambda s: pl.BlockSpec(s, lambda i: (0,) * len(s))
    bspec = lambda s: pl.BlockSpec(s, lambda i: (i, 0, 0))
    return pl.pallas_call(
        _mid_body,
        grid=(B,),
        in_specs=[bspec((1, N, W1)), bspec((1, N, W1)), bspec((1, N, 8)),
                  full((W1, W2)), full((W1, W2)), full((8, W2)), full((8, W2)),
                  full((1, W2))],
        out_specs=[bspec((1, N, W2)), bspec((1, N, W2))],
        out_shape=[jax.ShapeDtypeStruct((B, N, W2), jnp.float32),
                   jax.ShapeDtypeStruct((B, N, W2), jnp.float32)],
        interpret=interpret,
    )(a1, m1, coords, w2a, w2d, w2pn, w2pp, b2)


@functools.cache
def _make_gather_max(D):
    mesh = plsc.VectorSubcoreMesh(core_axis_name="c", subcore_axis_name="s")
    n_grp = PTS_PER_W // GRP

    @functools.partial(
        pl.kernel, mesh=mesh,
        out_type=jax.ShapeDtypeStruct((B * N, D), jnp.float32),
        scratch_types=[
            pltpu.VMEM((PTS_PER_W * K,), jnp.int32),
            pltpu.VMEM((GRP * K, D), jnp.float32),
            pltpu.VMEM((GRP * K, D), jnp.float32),
            pltpu.VMEM((GRP, D), jnp.float32),
            pltpu.VMEM((GRP, D), jnp.float32),
            pltpu.SemaphoreType.DMA,
            pltpu.SemaphoreType.DMA,
            pltpu.SemaphoreType.DMA,
            pltpu.SemaphoreType.DMA,
        ],
    )
    def gm(s_hbm, idx_hbm, out_hbm, idx_v, r0, r1, o0, o1, s0, s1, t0, t1):
        wid = lax.axis_index("s") * NC + lax.axis_index("c")
        base = wid * PTS_PER_W
        pltpu.sync_copy(idx_hbm.at[pl.ds(base * K, PTS_PER_W * K)], idx_v)
        bufs, sems = (r0, r1), (s0, s1)
        obufs, osems = (o0, o1), (t0, t1)

        def gather(g, buf, sem):
            ii = idx_v.at[pl.ds(g * (GRP * K), GRP * K)]
            return pltpu.make_async_copy(s_hbm.at[ii], buf, sem)

        def put(g, obuf, osem):
            return pltpu.make_async_copy(
                obuf, out_hbm.at[pl.ds(base + g * GRP, GRP)], osem)

        gather(0, r0, s0).start()
        gather(1, r1, s1).start()

        def body(i, _):
            for b in range(2):
                g = 2 * i + b
                buf, sem = bufs[b], sems[b]
                obuf, osem = obufs[b], osems[b]
                gather(g, buf, sem).wait()

                @pl.when(i > 0)
                def _w():
                    put(g, obuf, osem).wait()

                def point(p, _2):
                    r = p * K
                    for dc in range(D // 16):
                        sl = pl.ds(dc * 16, 16)
                        vals = [jnp.maximum(buf[r + j, sl], buf[r + j + 1, sl])
                                for j in range(0, K, 2)]
                        while len(vals) > 1:
                            vals = [jnp.maximum(vals[v], vals[v + 1])
                                    for v in range(0, len(vals), 2)]
                        obuf[p, sl] = vals[0]
                    return _2

                lax.fori_loop(0, GRP, point, None)
                put(g, obuf, osem).start()
                gather(jnp.minimum(g + 2, n_grp - 1), buf, sem).start()
            return _

        lax.fori_loop(0, n_grp // 2, body, None)
        gather(n_grp - 1, r0, s0).wait()
        gather(n_grp - 1, r1, s1).wait()
        put(n_grp - 2, o0, t0).wait()
        put(n_grp - 1, o1, t1).wait()

    return gm


def _head_body(a2_ref, m2_ref, x_ref, gw_ref, gb_ref, hwf_ref, hwg_ref,
               hb_ref, h2w_ref, h2b_ref, th_ref, sh_ref, sc_ref, out_ref):
    f2 = jnp.maximum(a2_ref[0] + m2_ref[0], 0.0)
    g = jnp.max(f2, axis=0, keepdims=True)
    g = jnp.maximum(jnp.dot(g, gw_ref[...],
                            preferred_element_type=jnp.float32, precision=lax.Precision.DEFAULT)
                    + gb_ref[...], 0.0)
    gh = jnp.dot(g, hwg_ref[...], preferred_element_type=jnp.float32, precision=lax.Precision.DEFAULT) \
        + hb_ref[...]
    h = jnp.maximum(jnp.dot(f2, hwf_ref[...],
                            preferred_element_type=jnp.float32, precision=lax.Precision.DEFAULT) + gh, 0.0)
    logits = jnp.dot(h, h2w_ref[...],
                     preferred_element_type=jnp.float32, precision=lax.Precision.DEFAULT) + h2b_ref[...]
    hag = x_ref[0][:, 3:4]
    t = sh_ref[0, 0] * (th_ref[0, 0] - hag)
    bias = sc_ref[0, 0] / (1.0 + jnp.exp(-t))
    cls = lax.broadcasted_iota(jnp.int32, (N, NUM_CLASSES), 1)
    out_ref[0] = jnp.where(cls == 0, logits + bias, logits)


def _head_call(a2, m2, x, gw, gb, hwf, hwg, hb, h2w, h2b, th, sh, sc,
               interpret=False):
    full = lambda s: pl.BlockSpec(s, lambda i: (0,) * len(s))
    bspec = lambda s: pl.BlockSpec(s, lambda i: (i, 0, 0))
    return pl.pallas_call(
        _head_body,
        grid=(B,),
        in_specs=[bspec((1, N, W2)), bspec((1, N, W2)), bspec((1, N, C_IN)),
                  full((W2, W2)), full((1, W2)),
                  full((W2, W2)), full((W2, W2)), full((1, W2)),
                  full((W2, NUM_CLASSES)), full((1, NUM_CLASSES)),
                  full((1, 1)), full((1, 1)), full((1, 1))],
        out_specs=bspec((1, N, NUM_CLASSES)),
        out_shape=jax.ShapeDtypeStruct((B, N, NUM_CLASSES), jnp.float32),
        interpret=interpret,
    )(a2, m2, x, gw, gb, hwf, hwg, hb, h2w, h2b, th, sh, sc)


def kernel(x, hmix_a, hmix_b, hmix_c, stem_W, stem_b, b1_W, b1_b,
           b2_W, b2_b, glob_W, glob_b, head1_W, head1_b, head2_W, head2_b,
           hp_thresh, hp_sharp, hp_scale):
    f32 = jnp.float32
    s11 = lambda v: jnp.asarray(v, f32).reshape(1, 1)
    pad8 = lambda w: jnp.concatenate(
        [w, jnp.zeros((8 - w.shape[0], w.shape[1]), f32)], axis=0)

    w1f, w1d, w1p = b1_W[:W0], b1_W[W0:2 * W0], b1_W[2 * W0:]
    w2f, w2d, w2p = b2_W[:W1], b2_W[W1:2 * W1], b2_W[2 * W1:]

    coords, a1, s1 = _prep_call(
        x, s11(hmix_a), s11(hmix_b), s11(hmix_c),
        stem_W, stem_b.reshape(1, W0),
        w1f - w1d, w1d, pad8(-w1p), pad8(w1p), b1_b.reshape(1, W1))

    coords_T = jnp.swapaxes(coords, 1, 2)
    ct_tiles = coords.reshape(B * (N // M_TILE), M_TILE, 8)
    idx = _topk_call(ct_tiles, coords_T)
    idx_flat = idx.reshape(B * N * K)

    m1 = _make_gather_max(W1)(s1.reshape(B * N, W1), idx_flat)

    a2, s2 = _mid_call(a1, m1.reshape(B, N, W1), coords,
                       w2f - w2d, w2d, pad8(-w2p), pad8(w2p),
                       b2_b.reshape(1, W2))

    m2 = _make_gather_max(W2)(s2.reshape(B * N, W2), idx_flat)

    return _head_call(a2, m2.reshape(B, N, W2), x,
                      glob_W, glob_b.reshape(1, W2),
                      head1_W[:W2], head1_W[W2:], head1_b.reshape(1, W2),
                      head2_W, head2_b.reshape(1, NUM_CLASSES),
                      s11(hp_thresh), s11(hp_sharp), s11(hp_scale))


# speedup vs baseline: 46.8184x; 1.1876x over previous
import functools

import jax
import jax.numpy as jnp
from jax import lax
from jax.experimental import pallas as pl
from jax.experimental.pallas import tpu as pltpu
from jax.experimental.pallas import tpu_sc as plsc

B, N, C_IN = 4, 4096, 4
K = 16
W0, W1, W2 = 64, 128, 256
NUM_CLASSES = 3

NC, NS = 2, 16
NW = NC * NS
PTS_PER_W = (B * N) // NW
GRP = 8
M_TILE = 256


def _prep_body(x_ref, a_ref, b_ref, c_ref, ws_ref, bs_ref,
               w1a_ref, w1d_ref, w1pn_ref, w1pp_ref, b1_ref,
               coords_ref, a1_ref, s1_ref):
    x = x_ref[0]
    a = a_ref[0, 0]
    b = b_ref[0, 0]
    c = c_ref[0, 0]
    z = a * x[:, 2:3] + b * x[:, 3:4] + c
    zeros = jnp.zeros((N, 5), jnp.float32)
    coords = jnp.concatenate([x[:, 0:2], z, zeros], axis=1)
    coords_ref[0] = coords
    f0 = jnp.maximum(jnp.dot(x, ws_ref[...],
                             preferred_element_type=jnp.float32, precision=lax.Precision.DEFAULT)
                     + bs_ref[...], 0.0)
    cp_p = jnp.dot(coords, w1pp_ref[...], preferred_element_type=jnp.float32, precision=lax.Precision.DEFAULT)
    cp_n = jnp.dot(coords, w1pn_ref[...], preferred_element_type=jnp.float32, precision=lax.Precision.DEFAULT)
    a1_ref[0] = (jnp.dot(f0, w1a_ref[...], preferred_element_type=jnp.float32, precision=lax.Precision.DEFAULT)
                 + cp_n + b1_ref[...])
    s1_ref[0] = (jnp.dot(f0, w1d_ref[...], preferred_element_type=jnp.float32, precision=lax.Precision.DEFAULT)
                 + cp_p)


def _prep_call(x, a, b, c, ws, bs, w1a, w1d, w1pn, w1pp, b1, interpret=False):
    full = lambda s: pl.BlockSpec(s, lambda i: (0,) * len(s))
    bspec = lambda s: pl.BlockSpec(s, lambda i: (i, 0, 0))
    return pl.pallas_call(
        _prep_body,
        grid=(B,),
        in_specs=[bspec((1, N, C_IN)), full((1, 1)), full((1, 1)), full((1, 1)),
                  full((C_IN, W0)), full((1, W0)),
                  full((W0, W1)), full((W0, W1)), full((8, W1)), full((8, W1)),
                  full((1, W1))],
        out_specs=[bspec((1, N, 8)), bspec((1, N, W1)), bspec((1, N, W1))],
        out_shape=[jax.ShapeDtypeStruct((B, N, 8), jnp.float32),
                   jax.ShapeDtypeStruct((B, N, W1), jnp.float32),
                   jax.ShapeDtypeStruct((B, N, W1), jnp.float32)],
        interpret=interpret,
    )(x, a, b, c, ws, bs, w1a, w1d, w1pn, w1pp, b1)


def _topk_body(ct_ref, cT_ref, idx_ref, *, prec):
    bi = pl.program_id(0)
    ct = ct_ref[0]
    cT = cT_ref[0]
    xx_t = jnp.sum(ct * ct, axis=1, keepdims=True)
    xx_f = jnp.sum(cT * cT, axis=0, keepdims=True)
    d = xx_t + xx_f - 2.0 * jnp.dot(ct, cT,
                                    preferred_element_type=jnp.float32, precision=prec)
    d = jnp.maximum(d, 0.0)
    NL = 128
    inf = jnp.float32(3.0e38)
    bigf = jnp.float32(1.0e9)
    lane = lax.broadcasted_iota(jnp.int32, (M_TILE, NL), 1).astype(jnp.float32)
    v = [jnp.full((M_TILE, NL), inf, jnp.float32) for _ in range(4)]
    for s in range(N // NL):
        x = lax.slice(d, (0, s * NL), (M_TILE, (s + 1) * NL)) + 1.0
        xb = lax.bitcast_convert_type(x, jnp.int32)
        t = lax.bitcast_convert_type((xb & jnp.int32(~31)) | jnp.int32(s),
                                     jnp.float32)
        lo = jnp.minimum(t, v[0])
        t = jnp.maximum(t, v[0])
        v[0] = lo
        lo = jnp.minimum(t, v[1])
        t = jnp.maximum(t, v[1])
        v[1] = lo
        lo = jnp.minimum(t, v[2])
        t = jnp.maximum(t, v[2])
        v[2] = lo
        v[3] = jnp.minimum(t, v[3])
    base = bi * N
    for k in range(K):
        m = jnp.min(v[0], axis=1, keepdims=True)
        lf = jnp.min(jnp.where(v[0] == m, lane, bigf), axis=1,
                     keepdims=True)
        slab = lax.bitcast_convert_type(m, jnp.int32) & jnp.int32(31)
        idx_ref[0, :, k:k + 1] = slab * NL + lf.astype(jnp.int32) + base
        eq = (v[0] == m) & (lane == lf)
        v = [jnp.where(eq, v[1], v[0]), jnp.where(eq, v[2], v[1]),
             jnp.where(eq, v[3], v[2]), jnp.where(eq, inf, v[3])]


def _topk_call(coords_tiles, coords_T, interpret=False,
               prec=lax.Precision.DEFAULT):
    n_t = N // M_TILE
    return pl.pallas_call(
        functools.partial(_topk_body, prec=prec),
        grid=(B, n_t),
        in_specs=[pl.BlockSpec((1, M_TILE, 8), lambda bi, t: (bi * n_t + t, 0, 0)),
                  pl.BlockSpec((1, 8, N), lambda bi, t: (bi, 0, 0))],
        out_specs=pl.BlockSpec((1, M_TILE, K), lambda bi, t: (bi * n_t + t, 0, 0)),
        out_shape=jax.ShapeDtypeStruct((B * n_t, M_TILE, K), jnp.int32),
        interpret=interpret,
    )(coords_tiles, coords_T)


def _mid_body(a1_ref, m1_ref, coords_ref, w2a_ref, w2d_ref, w2pn_ref,
              w2pp_ref, b2_ref, a2_ref, s2_ref):
    f1 = jnp.maximum(a1_ref[0] + m1_ref[0], 0.0)
    coords = coords_ref[0]
    cp_p = jnp.dot(coords, w2pp_ref[...], preferred_element_type=jnp.float32, precision=lax.Precision.DEFAULT)
    cp_n = jnp.dot(coords, w2pn_ref[...], preferred_element_type=jnp.float32, precision=lax.Precision.DEFAULT)
    a2_ref[0] = (jnp.dot(f1, w2a_ref[...], preferred_element_type=jnp.float32, precision=lax.Precision.DEFAULT)
                 + cp_n + b2_ref[...])
    s2_ref[0] = (jnp.dot(f1, w2d_ref[...], preferred_element_type=jnp.float32, precision=lax.Precision.DEFAULT)
                 + cp_p)


def _mid_call(a1, m1, coords, w2a, w2d, w2pn, w2pp, b2, interpret=False):
    full = lambda s: pl.BlockSpec(s, lambda i: (0,) * len(s))
    bspec = lambda s: pl.BlockSpec(s, lambda i: (i, 0, 0))
    return pl.pallas_call(
        _mid_body,
        grid=(B,),
        in_specs=[bspec((1, N, W1)), bspec((1, N, W1)), bspec((1, N, 8)),
                  full((W1, W2)), full((W1, W2)), full((8, W2)), full((8, W2)),
                  full((1, W2))],
        out_specs=[bspec((1, N, W2)), bspec((1, N, W2))],
        out_shape=[jax.ShapeDtypeStruct((B, N, W2), jnp.float32),
                   jax.ShapeDtypeStruct((B, N, W2), jnp.float32)],
        interpret=interpret,
    )(a1, m1, coords, w2a, w2d, w2pn, w2pp, b2)


@functools.cache
def _make_gather_max(D):
    mesh = plsc.VectorSubcoreMesh(core_axis_name="c", subcore_axis_name="s")
    n_grp = PTS_PER_W // GRP

    @functools.partial(
        pl.kernel, mesh=mesh,
        out_type=jax.ShapeDtypeStruct((B * N, D), jnp.float32),
        scratch_types=[
            pltpu.VMEM((PTS_PER_W * K,), jnp.int32),
            pltpu.VMEM((GRP * K, D), jnp.float32),
            pltpu.VMEM((GRP * K, D), jnp.float32),
            pltpu.VMEM((GRP, D), jnp.float32),
            pltpu.VMEM((GRP, D), jnp.float32),
            pltpu.SemaphoreType.DMA,
            pltpu.SemaphoreType.DMA,
            pltpu.SemaphoreType.DMA,
            pltpu.SemaphoreType.DMA,
        ],
    )
    def gm(s_hbm, idx_hbm, out_hbm, idx_v, r0, r1, o0, o1, s0, s1, t0, t1):
        wid = lax.axis_index("s") * NC + lax.axis_index("c")
        base = wid * PTS_PER_W
        pltpu.sync_copy(idx_hbm.at[pl.ds(base * K, PTS_PER_W * K)], idx_v)
        bufs, sems = (r0, r1), (s0, s1)
        obufs, osems = (o0, o1), (t0, t1)

        def gather(g, buf, sem):
            ii = idx_v.at[pl.ds(g * (GRP * K), GRP * K)]
            return pltpu.make_async_copy(s_hbm.at[ii], buf, sem)

        def put(g, obuf, osem):
            return pltpu.make_async_copy(
                obuf, out_hbm.at[pl.ds(base + g * GRP, GRP)], osem)

        gather(0, r0, s0).start()
        gather(1, r1, s1).start()

        def body(i, _):
            for b in range(2):
                g = 2 * i + b
                buf, sem = bufs[b], sems[b]
                obuf, osem = obufs[b], osems[b]
                gather(g, buf, sem).wait()

                @pl.when(i > 0)
                def _w():
                    put(g, obuf, osem).wait()

                def point(p, _2):
                    r = p * K
                    for dc in range(D // 16):
                        sl = pl.ds(dc * 16, 16)
                        vals = [jnp.maximum(buf[r + j, sl], buf[r + j + 1, sl])
                                for j in range(0, K, 2)]
                        while len(vals) > 1:
                            vals = [jnp.maximum(vals[v], vals[v + 1])
                                    for v in range(0, len(vals), 2)]
                        obuf[p, sl] = vals[0]
                    return _2

                lax.fori_loop(0, GRP, point, None)
                put(g, obuf, osem).start()
                gather(jnp.minimum(g + 2, n_grp - 1), buf, sem).start()
            return _

        lax.fori_loop(0, n_grp // 2, body, None)
        gather(n_grp - 1, r0, s0).wait()
        gather(n_grp - 1, r1, s1).wait()
        put(n_grp - 2, o0, t0).wait()
        put(n_grp - 1, o1, t1).wait()

    return gm


def _head_body(a2_ref, m2_ref, x_ref, gw_ref, gb_ref, hwf_ref, hwg_ref,
               hb_ref, h2w_ref, h2b_ref, th_ref, sh_ref, sc_ref, out_ref):
    f2 = jnp.maximum(a2_ref[0] + m2_ref[0], 0.0)
    g = jnp.max(f2, axis=0, keepdims=True)
    g = jnp.maximum(jnp.dot(g, gw_ref[...],
                            preferred_element_type=jnp.float32, precision=lax.Precision.DEFAULT)
                    + gb_ref[...], 0.0)
    gh = jnp.dot(g, hwg_ref[...], preferred_element_type=jnp.float32, precision=lax.Precision.DEFAULT) \
        + hb_ref[...]
    h = jnp.maximum(jnp.dot(f2, hwf_ref[...],
                            preferred_element_type=jnp.float32, precision=lax.Precision.DEFAULT) + gh, 0.0)
    logits = jnp.dot(h, h2w_ref[...],
                     preferred_element_type=jnp.float32, precision=lax.Precision.DEFAULT) + h2b_ref[...]
    hag = x_ref[0][:, 3:4]
    t = sh_ref[0, 0] * (th_ref[0, 0] - hag)
    bias = sc_ref[0, 0] / (1.0 + jnp.exp(-t))
    cls = lax.broadcasted_iota(jnp.int32, (N, NUM_CLASSES), 1)
    out_ref[0] = jnp.where(cls == 0, logits + bias, logits)


def _head_call(a2, m2, x, gw, gb, hwf, hwg, hb, h2w, h2b, th, sh, sc,
               interpret=False):
    full = lambda s: pl.BlockSpec(s, lambda i: (0,) * len(s))
    bspec = lambda s: pl.BlockSpec(s, lambda i: (i, 0, 0))
    return pl.pallas_call(
        _head_body,
        grid=(B,),
        in_specs=[bspec((1, N, W2)), bspec((1, N, W2)), bspec((1, N, C_IN)),
                  full((W2, W2)), full((1, W2)),
                  full((W2, W2)), full((W2, W2)), full((1, W2)),
                  full((W2, NUM_CLASSES)), full((1, NUM_CLASSES)),
                  full((1, 1)), full((1, 1)), full((1, 1))],
        out_specs=bspec((1, N, NUM_CLASSES)),
        out_shape=jax.ShapeDtypeStruct((B, N, NUM_CLASSES), jnp.float32),
        interpret=interpret,
    )(a2, m2, x, gw, gb, hwf, hwg, hb, h2w, h2b, th, sh, sc)


def kernel(x, hmix_a, hmix_b, hmix_c, stem_W, stem_b, b1_W, b1_b,
           b2_W, b2_b, glob_W, glob_b, head1_W, head1_b, head2_W, head2_b,
           hp_thresh, hp_sharp, hp_scale):
    f32 = jnp.float32
    s11 = lambda v: jnp.asarray(v, f32).reshape(1, 1)
    pad8 = lambda w: jnp.concatenate(
        [w, jnp.zeros((8 - w.shape[0], w.shape[1]), f32)], axis=0)

    w1f, w1d, w1p = b1_W[:W0], b1_W[W0:2 * W0], b1_W[2 * W0:]
    w2f, w2d, w2p = b2_W[:W1], b2_W[W1:2 * W1], b2_W[2 * W1:]

    coords, a1, s1 = _prep_call(
        x, s11(hmix_a), s11(hmix_b), s11(hmix_c),
        stem_W, stem_b.reshape(1, W0),
        w1f - w1d, w1d, pad8(-w1p), pad8(w1p), b1_b.reshape(1, W1))

    coords_T = jnp.swapaxes(coords, 1, 2)
    ct_tiles = coords.reshape(B * (N // M_TILE), M_TILE, 8)
    idx = _topk_call(ct_tiles, coords_T)
    idx_flat = idx.reshape(B * N * K)

    m1 = _make_gather_max(W1)(s1.reshape(B * N, W1), idx_flat)

    a2, s2 = _mid_call(a1, m1.reshape(B, N, W1), coords,
                       w2f - w2d, w2d, pad8(-w2p), pad8(w2p),
                       b2_b.reshape(1, W2))

    m2 = _make_gather_max(W2)(s2.reshape(B * N, W2), idx_flat)

    return _head_call(a2, m2.reshape(B, N, W2), x,
                      glob_W, glob_b.reshape(1, W2),
                      head1_W[:W2], head1_W[W2:], head1_b.reshape(1, W2),
                      head2_W, head2_b.reshape(1, NUM_CLASSES),
                      s11(hp_thresh), s11(hp_sharp), s11(hp_scale))
